# Initial kernel scaffold; baseline (speedup 1.0000x reference)
import jax
import jax.numpy as jnp
from jax.experimental import pallas as pl


def kernel(loc, vel, h, edge_index, W_emb, b_emb, We1, be1, We2, be2, Wc1, bc1, Wc2, Wn1, bn1, Wn2, bn2, Wv, bv, Wm1, bm1, Wm2, bm2, Wm3, bm3, log_std):
    raise NotImplementedError("write your pallas kernel here")


# speedup vs baseline: 9.6885x; 9.6885x over previous
import dataclasses

import jax
import jax.numpy as jnp
from jax import lax
from jax.experimental import pallas as pl
from jax.experimental.pallas import tpu as pltpu
from jax.experimental.pallas import tpu_sc as plsc

N = 10000
E = 640000
HID = 64
TBL = 128
NC, NS = 2, 16
NW = NC * NS
EW = E // NW
CG = 400
CS = 2000
BE = 5120
NP = 10240
PW = 3 * NP

_F32 = jnp.float32


def _sc_params():
    cp = pltpu.CompilerParams()
    if "needs_layout_passes" in pltpu.CompilerParams.__dataclass_fields__:
        cp = dataclasses.replace(cp, needs_layout_passes=False)
    return cp


def _node_pre_body(h_ref, loc_ref, wemb_ref, bemb_ref, we1a_ref, we1b_ref,
                   be1_ref, wv_ref, bv_ref, t1_ref, t2_ref, g_ref):
    hh = jnp.dot(h_ref[...], wemb_ref[...], preferred_element_type=_F32) + bemb_ref[...]
    a = jnp.dot(hh, we1a_ref[...], preferred_element_type=_F32) + be1_ref[...]
    b = jnp.dot(hh, we1b_ref[...], preferred_element_type=_F32)
    loc = loc_ref[...]
    pad = jnp.zeros((N, TBL - HID - 2), _F32)
    t1_ref[...] = jnp.concatenate([a, loc, pad], axis=1)
    t2_ref[...] = jnp.concatenate([b, -loc, pad], axis=1)
    g_ref[...] = jnp.dot(hh, wv_ref[...], preferred_element_type=_F32) + bv_ref[...]


def _node_pre(h, loc, W_emb, b_emb, We1a, We1b, be1, Wv, bv):
    return pl.pallas_call(
        _node_pre_body,
        out_shape=[
            jax.ShapeDtypeStruct((N, TBL), _F32),
            jax.ShapeDtypeStruct((N, TBL), _F32),
            jax.ShapeDtypeStruct((N, 1), _F32),
        ],
    )(h, loc, W_emb, b_emb, We1a, We1b, be1, Wv, bv)


def _sc_gather_body(t1_hbm, t2_hbm, rows_hbm, cols_hbm, x1_hbm, x2_hbm,
                    idx1, idx2, buf1, buf2, sem1, sem2):
    wid = lax.axis_index("s") * NC + lax.axis_index("c")
    base0 = wid * EW

    @pl.loop(0, EW, step=CG)
    def _(i):
        base = base0 + i
        pltpu.sync_copy(rows_hbm.at[pl.ds(base, CG)], idx1)
        pltpu.sync_copy(cols_hbm.at[pl.ds(base, CG)], idx2)
        c1 = pltpu.async_copy(t1_hbm.at[idx1], buf1, sem1)
        c2 = pltpu.async_copy(t2_hbm.at[idx2], buf2, sem2)
        c1.wait()
        c2.wait()
        pltpu.sync_copy(buf1, x1_hbm.at[pl.ds(base, CG)])
        pltpu.sync_copy(buf2, x2_hbm.at[pl.ds(base, CG)])


def _sc_gather(T1, T2, rows, cols):
    return pl.kernel(
        _sc_gather_body,
        out_type=[
            jax.ShapeDtypeStruct((E, TBL), _F32),
            jax.ShapeDtypeStruct((E, TBL), _F32),
        ],
        mesh=plsc.VectorSubcoreMesh(core_axis_name="c", subcore_axis_name="s"),
        scratch_types=[
            pltpu.VMEM((CG,), jnp.int32),
            pltpu.VMEM((CG,), jnp.int32),
            pltpu.VMEM((CG, TBL), _F32),
            pltpu.VMEM((CG, TBL), _F32),
            pltpu.SemaphoreType.DMA,
            pltpu.SemaphoreType.DMA,
        ],
    )(T1, T2, rows, cols)


def _edge_mlp_body(x1_ref, x2_ref, we2_ref, be2_ref, wc1_ref, bc1_ref,
                   wc2t_ref, w2p_ref, tx_ref, ty_ref):
    x = x1_ref[...] + x2_ref[...]
    x64 = x[:, :HID]
    diff = x[:, HID:HID + 2]
    radial = jnp.sum(diff * diff, axis=1, keepdims=True)
    w2sum = w2p_ref[0:1, :] + w2p_ref[1:2, :]
    u = jax.nn.silu(x64 + radial * w2sum)
    m = jax.nn.silu(jnp.dot(u, we2_ref[...], preferred_element_type=_F32) + be2_ref[...])
    p = jax.nn.silu(jnp.dot(m, wc1_ref[...], preferred_element_type=_F32) + bc1_ref[...])
    s = jnp.sum(p * wc2t_ref[...], axis=1, keepdims=True)
    coord = diff / (jnp.sqrt(radial) + 1.0)
    trans = coord * s
    tx_ref[...] = trans[:, 0].reshape(BE // 128, 128)
    ty_ref[...] = trans[:, 1].reshape(BE // 128, 128)


def _edge_mlp(X1, X2, We2, be2, Wc1, bc1, Wc2t, w2p):
    wspec = lambda shp: pl.BlockSpec(shp, lambda i: (0, 0))
    return pl.pallas_call(
        _edge_mlp_body,
        grid=(E // BE,),
        in_specs=[
            pl.BlockSpec((BE, TBL), lambda i: (i, 0)),
            pl.BlockSpec((BE, TBL), lambda i: (i, 0)),
            wspec((HID, HID)),
            wspec((1, HID)),
            wspec((HID, HID)),
            wspec((1, HID)),
            wspec((1, HID)),
            wspec((2, HID)),
        ],
        out_specs=[
            pl.BlockSpec((BE // 128, 128), lambda i: (i, 0)),
            pl.BlockSpec((BE // 128, 128), lambda i: (i, 0)),
        ],
        out_shape=[
            jax.ShapeDtypeStruct((E // 128, 128), _F32),
            jax.ShapeDtypeStruct((E // 128, 128), _F32),
        ],
    )(X1, X2, We2, be2, Wc1, bc1, Wc2t, w2p)


def _sc_scatter_body(tx_hbm, ty_hbm, rows_hbm, out_hbm,
                     ibuf, xbuf, ybuf, accx, accy, accc):
    wid = lax.axis_index("s") * NC + lax.axis_index("c")
    base0 = wid * EW
    zf = jnp.zeros((16,), _F32)
    onesf = jnp.full((16,), 1.0, _F32)

    @pl.loop(0, NP, step=16)
    def _(j):
        accx[pl.ds(j, 16)] = zf
        accy[pl.ds(j, 16)] = zf
        accc[pl.ds(j, 16)] = zf

    @pl.loop(0, EW, step=CS)
    def _(i):
        base = base0 + i
        pltpu.sync_copy(rows_hbm.at[pl.ds(base, CS)], ibuf)
        pltpu.sync_copy(tx_hbm.at[pl.ds(base, CS)], xbuf)
        pltpu.sync_copy(ty_hbm.at[pl.ds(base, CS)], ybuf)

        @pl.loop(0, CS, step=16)
        def _(j):
            r16 = ibuf[pl.ds(j, 16)]
            plsc.addupdate_scatter(accx, [r16], xbuf[pl.ds(j, 16)])
            plsc.addupdate_scatter(accy, [r16], ybuf[pl.ds(j, 16)])
            plsc.addupdate_scatter(accc, [r16], onesf)

    obase = wid * PW
    pltpu.sync_copy(accx, out_hbm.at[pl.ds(obase, NP)])
    pltpu.sync_copy(accy, out_hbm.at[pl.ds(obase + NP, NP)])
    pltpu.sync_copy(accc, out_hbm.at[pl.ds(obase + 2 * NP, NP)])


def _sc_scatter(tx, ty, rows):
    return pl.kernel(
        _sc_scatter_body,
        out_type=jax.ShapeDtypeStruct((NW * PW,), _F32),
        mesh=plsc.VectorSubcoreMesh(core_axis_name="c", subcore_axis_name="s"),
        scratch_types=[
            pltpu.VMEM((CS,), jnp.int32),
            pltpu.VMEM((CS,), _F32),
            pltpu.VMEM((CS,), _F32),
            pltpu.VMEM((NP,), _F32),
            pltpu.VMEM((NP,), _F32),
            pltpu.VMEM((NP,), _F32),
        ],
        compiler_params=_sc_params(),
    )(tx, ty, rows)


def _psum_body(part_ref, out_ref):
    out_ref[...] = jnp.sum(part_ref[...], axis=0)


def _psum(partials):
    return pl.pallas_call(
        _psum_body,
        out_shape=jax.ShapeDtypeStruct((3 * (NP // 128), 128), _F32),
    )(partials)


def _final_body(ax_ref, ay_ref, ac_ref, g_ref, vel_ref, loc_ref, h_ref,
                wm1_ref, bm1_ref, wm2_ref, bm2_ref, wm3_ref, bm3_ref,
                ls_ref, mu_ref, lp_ref):
    ax = ax_ref[...]
    ay = ay_ref[...]
    cnt = jnp.maximum(ac_ref[...], 1.0)
    agg = jnp.concatenate([ax / cnt, ay / cnt], axis=1)
    vel = vel_ref[...]
    vel_pred = g_ref[...] * vel + agg
    z = jnp.concatenate([loc_ref[...], vel, h_ref[...]], axis=1)
    z = jax.nn.silu(jnp.dot(z, wm1_ref[...], preferred_element_type=_F32) + bm1_ref[...])
    z = jax.nn.silu(jnp.dot(z, wm2_ref[...], preferred_element_type=_F32) + bm2_ref[...])
    mu = vel_pred + jnp.dot(z, wm3_ref[...], preferred_element_type=_F32) + bm3_ref[...]
    mu_ref[...] = mu
    lp_ref[...] = jnp.zeros(mu.shape, _F32) - ls_ref[...] - 0.5 * jnp.log(2.0 * jnp.pi)


BN = 2000


def _final(ax, ay, ac, g, vel, loc, h, Wm1, bm1, Wm2, bm2, Wm3, bm3, log_std):
    nspec = lambda c: pl.BlockSpec((BN, c), lambda i: (i, 0))
    wspec = lambda shp: pl.BlockSpec(shp, lambda i: (0, 0))
    return pl.pallas_call(
        _final_body,
        grid=(N // BN,),
        in_specs=[
            nspec(1), nspec(1), nspec(1), nspec(1), nspec(2), nspec(2),
            nspec(30),
            wspec((34, 128)), wspec((1, 128)), wspec((128, 128)),
            wspec((1, 128)), wspec((128, 2)), wspec((1, 2)), wspec((1, 2)),
        ],
        out_specs=[
            pl.BlockSpec((BN, 2), lambda i: (i, 0)),
            pl.BlockSpec((BN, 2), lambda i: (i, 0)),
        ],
        out_shape=[
            jax.ShapeDtypeStruct((N, 2), _F32),
            jax.ShapeDtypeStruct((N, 2), _F32),
        ],
    )(ax, ay, ac, g, vel, loc, h, Wm1, bm1, Wm2, bm2, Wm3, bm3, log_std)


def kernel(loc, vel, h, edge_index, W_emb, b_emb, We1, be1, We2, be2, Wc1,
           bc1, Wc2, Wn1, bn1, Wn2, bn2, Wv, bv, Wm1, bm1, Wm2, bm2, Wm3,
           bm3, log_std):
    rows = edge_index[0]
    cols = edge_index[1]

    T1, T2, g = _node_pre(h, loc, W_emb, b_emb.reshape(1, HID),
                          We1[:HID], We1[HID:2 * HID], be1.reshape(1, HID),
                          Wv, bv.reshape(1, 1))
    X1, X2 = _sc_gather(T1, T2, rows, cols)
    tx2d, ty2d = _edge_mlp(X1, X2, We2, be2.reshape(1, HID), Wc1,
                           bc1.reshape(1, HID), Wc2.reshape(1, HID),
                           We1[2 * HID:2 * HID + 2])
    partials = _sc_scatter(tx2d.reshape(E), ty2d.reshape(E), rows)
    psum = _psum(partials.reshape(NW, 3 * (NP // 128), 128))
    f = psum.reshape(3, NP)
    ax = f[0, :N].reshape(N, 1)
    ay = f[1, :N].reshape(N, 1)
    ac = f[2, :N].reshape(N, 1)
    mu, logp = _final(ax, ay, ac, g, vel,
                      loc, h, Wm1, bm1.reshape(1, 128), Wm2,
                      bm2.reshape(1, 128), Wm3, bm3.reshape(1, 2),
                      log_std.reshape(1, 2))
    return (mu.reshape(100, 100, 2), logp.reshape(100, 100, 2))


# speedup vs baseline: 13.4195x; 1.3851x over previous
import dataclasses

import jax
import jax.numpy as jnp
from jax import lax
from jax.experimental import pallas as pl
from jax.experimental.pallas import tpu as pltpu
from jax.experimental.pallas import tpu_sc as plsc

N = 10000
E = 640000
HID = 64
TBL = 128
NC, NS = 2, 16
NW = NC * NS
EW = E // NW
K = 5
EK = E // K
EWK = EK // NW
CG = 400
CS = 2000
BE = 5120
NP = 10240
PW = 3 * NP

_F32 = jnp.float32


def _sc_params():
    cp = pltpu.CompilerParams()
    if "needs_layout_passes" in pltpu.CompilerParams.__dataclass_fields__:
        cp = dataclasses.replace(cp, needs_layout_passes=False)
    return cp


def _node_pre_body(h_ref, loc_ref, wemb_ref, bemb_ref, we1a_ref, we1b_ref,
                   be1_ref, wv_ref, bv_ref, t1_ref, t2_ref, g_ref):
    hh = jnp.dot(h_ref[...], wemb_ref[...], preferred_element_type=_F32) + bemb_ref[...]
    a = jnp.dot(hh, we1a_ref[...], preferred_element_type=_F32) + be1_ref[...]
    b = jnp.dot(hh, we1b_ref[...], preferred_element_type=_F32)
    loc = loc_ref[...]
    pad = jnp.zeros((N, TBL - HID - 2), _F32)
    t1_ref[...] = jnp.concatenate([a, loc, pad], axis=1)
    t2_ref[...] = jnp.concatenate([b, -loc, pad], axis=1)
    g_ref[...] = jnp.dot(hh, wv_ref[...], preferred_element_type=_F32) + bv_ref[...]


def _node_pre(h, loc, W_emb, b_emb, We1a, We1b, be1, Wv, bv):
    return pl.pallas_call(
        _node_pre_body,
        out_shape=[
            jax.ShapeDtypeStruct((N, TBL), _F32),
            jax.ShapeDtypeStruct((N, TBL), _F32),
            jax.ShapeDtypeStruct((N, 1), _F32),
        ],
    )(h, loc, W_emb, b_emb, We1a, We1b, be1, Wv, bv)


def _sc_gather_body(t1_hbm, t2_hbm, rows_hbm, cols_hbm, x1_hbm, x2_hbm,
                    idx1, idx2, buf1, buf2, sem1, sem2):
    wid = lax.axis_index("s") * NC + lax.axis_index("c")
    base0 = wid * EWK

    @pl.loop(0, EWK, step=CG)
    def _(i):
        base = base0 + i
        pltpu.sync_copy(rows_hbm.at[pl.ds(base, CG)], idx1)
        pltpu.sync_copy(cols_hbm.at[pl.ds(base, CG)], idx2)
        c1 = pltpu.async_copy(t1_hbm.at[idx1], buf1, sem1)
        c2 = pltpu.async_copy(t2_hbm.at[idx2], buf2, sem2)
        c1.wait()
        c2.wait()
        pltpu.sync_copy(buf1, x1_hbm.at[pl.ds(base, CG)])
        pltpu.sync_copy(buf2, x2_hbm.at[pl.ds(base, CG)])


def _sc_gather(T1, T2, rows, cols):
    return pl.kernel(
        _sc_gather_body,
        out_type=[
            jax.ShapeDtypeStruct((EK, TBL), _F32),
            jax.ShapeDtypeStruct((EK, TBL), _F32),
        ],
        mesh=plsc.VectorSubcoreMesh(core_axis_name="c", subcore_axis_name="s"),
        scratch_types=[
            pltpu.VMEM((CG,), jnp.int32),
            pltpu.VMEM((CG,), jnp.int32),
            pltpu.VMEM((CG, TBL), _F32),
            pltpu.VMEM((CG, TBL), _F32),
            pltpu.SemaphoreType.DMA,
            pltpu.SemaphoreType.DMA,
        ],
    )(T1, T2, rows, cols)


def _edge_mlp_body(x1_ref, x2_ref, we2_ref, be2_ref, wc1_ref, bc1_ref,
                   wc2t_ref, w2p_ref, tx_ref, ty_ref):
    x = x1_ref[...] + x2_ref[...]
    x64 = x[:, :HID]
    diff = x[:, HID:HID + 2]
    radial = jnp.sum(diff * diff, axis=1, keepdims=True)
    w2sum = w2p_ref[0:1, :] + w2p_ref[1:2, :]
    u = jax.nn.silu(x64 + radial * w2sum)
    m = jax.nn.silu(jnp.dot(u, we2_ref[...], preferred_element_type=_F32) + be2_ref[...])
    p = jax.nn.silu(jnp.dot(m, wc1_ref[...], preferred_element_type=_F32) + bc1_ref[...])
    s = jnp.sum(p * wc2t_ref[...], axis=1, keepdims=True)
    coord = diff / (jnp.sqrt(radial) + 1.0)
    trans = coord * s
    tx_ref[...] = trans[:, 0].reshape(BE // 128, 128)
    ty_ref[...] = trans[:, 1].reshape(BE // 128, 128)


def _edge_mlp(X1, X2, We2, be2, Wc1, bc1, Wc2t, w2p):
    wspec = lambda shp: pl.BlockSpec(shp, lambda i: (0, 0))
    return pl.pallas_call(
        _edge_mlp_body,
        grid=(EK // BE,),
        in_specs=[
            pl.BlockSpec((BE, TBL), lambda i: (i, 0)),
            pl.BlockSpec((BE, TBL), lambda i: (i, 0)),
            wspec((HID, HID)),
            wspec((1, HID)),
            wspec((HID, HID)),
            wspec((1, HID)),
            wspec((1, HID)),
            wspec((2, HID)),
        ],
        out_specs=[
            pl.BlockSpec((BE // 128, 128), lambda i: (i, 0)),
            pl.BlockSpec((BE // 128, 128), lambda i: (i, 0)),
        ],
        out_shape=[
            jax.ShapeDtypeStruct((EK // 128, 128), _F32),
            jax.ShapeDtypeStruct((EK // 128, 128), _F32),
        ],
    )(X1, X2, We2, be2, Wc1, bc1, Wc2t, w2p)


def _sc_scatter_body(tx_hbm, ty_hbm, rows_hbm, out_hbm,
                     ibuf, xbuf, ybuf, accx, accy, accc):
    wid = lax.axis_index("s") * NC + lax.axis_index("c")
    base0 = wid * EW
    zf = jnp.zeros((16,), _F32)
    onesf = jnp.full((16,), 1.0, _F32)

    @pl.loop(0, NP, step=16)
    def _(j):
        accx[pl.ds(j, 16)] = zf
        accy[pl.ds(j, 16)] = zf
        accc[pl.ds(j, 16)] = zf

    @pl.loop(0, EW, step=CS)
    def _(i):
        base = base0 + i
        pltpu.sync_copy(rows_hbm.at[pl.ds(base, CS)], ibuf)
        pltpu.sync_copy(tx_hbm.at[pl.ds(base, CS)], xbuf)
        pltpu.sync_copy(ty_hbm.at[pl.ds(base, CS)], ybuf)

        @pl.loop(0, CS, step=16)
        def _(j):
            r16 = ibuf[pl.ds(j, 16)]
            plsc.addupdate_scatter(accx, [r16], xbuf[pl.ds(j, 16)])
            plsc.addupdate_scatter(accy, [r16], ybuf[pl.ds(j, 16)])
            plsc.addupdate_scatter(accc, [r16], onesf)

    obase = wid * PW
    pltpu.sync_copy(accx, out_hbm.at[pl.ds(obase, NP)])
    pltpu.sync_copy(accy, out_hbm.at[pl.ds(obase + NP, NP)])
    pltpu.sync_copy(accc, out_hbm.at[pl.ds(obase + 2 * NP, NP)])


def _sc_scatter(tx, ty, rows):
    return pl.kernel(
        _sc_scatter_body,
        out_type=jax.ShapeDtypeStruct((NW * PW,), _F32),
        mesh=plsc.VectorSubcoreMesh(core_axis_name="c", subcore_axis_name="s"),
        scratch_types=[
            pltpu.VMEM((CS,), jnp.int32),
            pltpu.VMEM((CS,), _F32),
            pltpu.VMEM((CS,), _F32),
            pltpu.VMEM((NP,), _F32),
            pltpu.VMEM((NP,), _F32),
            pltpu.VMEM((NP,), _F32),
        ],
        compiler_params=_sc_params(),
    )(tx, ty, rows)


def _psum_body(part_ref, out_ref):
    out_ref[...] = jnp.sum(part_ref[...], axis=0)


def _psum(partials):
    return pl.pallas_call(
        _psum_body,
        out_shape=jax.ShapeDtypeStruct((3 * (NP // 128), 128), _F32),
    )(partials)


def _final_body(ax_ref, ay_ref, ac_ref, g_ref, vel_ref, loc_ref, h_ref,
                wm1_ref, bm1_ref, wm2_ref, bm2_ref, wm3_ref, bm3_ref,
                ls_ref, mu_ref, lp_ref):
    ax = ax_ref[...]
    ay = ay_ref[...]
    cnt = jnp.maximum(ac_ref[...], 1.0)
    agg = jnp.concatenate([ax / cnt, ay / cnt], axis=1)
    vel = vel_ref[...]
    vel_pred = g_ref[...] * vel + agg
    z = jnp.concatenate([loc_ref[...], vel, h_ref[...]], axis=1)
    z = jax.nn.silu(jnp.dot(z, wm1_ref[...], preferred_element_type=_F32) + bm1_ref[...])
    z = jax.nn.silu(jnp.dot(z, wm2_ref[...], preferred_element_type=_F32) + bm2_ref[...])
    mu = vel_pred + jnp.dot(z, wm3_ref[...], preferred_element_type=_F32) + bm3_ref[...]
    mu_ref[...] = mu
    lp_ref[...] = jnp.zeros(mu.shape, _F32) - ls_ref[...] - 0.5 * jnp.log(2.0 * jnp.pi)


BN = 2000


def _final(ax, ay, ac, g, vel, loc, h, Wm1, bm1, Wm2, bm2, Wm3, bm3, log_std):
    nspec = lambda c: pl.BlockSpec((BN, c), lambda i: (i, 0))
    wspec = lambda shp: pl.BlockSpec(shp, lambda i: (0, 0))
    return pl.pallas_call(
        _final_body,
        grid=(N // BN,),
        in_specs=[
            nspec(1), nspec(1), nspec(1), nspec(1), nspec(2), nspec(2),
            nspec(30),
            wspec((34, 128)), wspec((1, 128)), wspec((128, 128)),
            wspec((1, 128)), wspec((128, 2)), wspec((1, 2)), wspec((1, 2)),
        ],
        out_specs=[
            pl.BlockSpec((BN, 2), lambda i: (i, 0)),
            pl.BlockSpec((BN, 2), lambda i: (i, 0)),
        ],
        out_shape=[
            jax.ShapeDtypeStruct((N, 2), _F32),
            jax.ShapeDtypeStruct((N, 2), _F32),
        ],
    )(ax, ay, ac, g, vel, loc, h, Wm1, bm1, Wm2, bm2, Wm3, bm3, log_std)


def kernel(loc, vel, h, edge_index, W_emb, b_emb, We1, be1, We2, be2, Wc1,
           bc1, Wc2, Wn1, bn1, Wn2, bn2, Wv, bv, Wm1, bm1, Wm2, bm2, Wm3,
           bm3, log_std):
    rows = edge_index[0]
    cols = edge_index[1]

    T1, T2, g = _node_pre(h, loc, W_emb, b_emb.reshape(1, HID),
                          We1[:HID], We1[HID:2 * HID], be1.reshape(1, HID),
                          Wv, bv.reshape(1, 1))
    txs, tys = [], []
    for k in range(K):
        sl = slice(k * EK, (k + 1) * EK)
        X1, X2 = _sc_gather(T1, T2, rows[sl], cols[sl])
        txk, tyk = _edge_mlp(X1, X2, We2, be2.reshape(1, HID), Wc1,
                             bc1.reshape(1, HID), Wc2.reshape(1, HID),
                             We1[2 * HID:2 * HID + 2])
        txs.append(txk)
        tys.append(tyk)
    tx2d = jnp.concatenate(txs, axis=0)
    ty2d = jnp.concatenate(tys, axis=0)
    partials = _sc_scatter(tx2d.reshape(E), ty2d.reshape(E), rows)
    psum = _psum(partials.reshape(NW, 3 * (NP // 128), 128))
    f = psum.reshape(3, NP)
    ax = f[0, :N].reshape(N, 1)
    ay = f[1, :N].reshape(N, 1)
    ac = f[2, :N].reshape(N, 1)
    mu, logp = _final(ax, ay, ac, g, vel,
                      loc, h, Wm1, bm1.reshape(1, 128), Wm2,
                      bm2.reshape(1, 128), Wm3, bm3.reshape(1, 2),
                      log_std.reshape(1, 2))
    return (mu.reshape(100, 100, 2), logp.reshape(100, 100, 2))


# speedup vs baseline: 14.2202x; 1.0597x over previous
import dataclasses

import jax
import jax.numpy as jnp
from jax import lax
from jax.experimental import pallas as pl
from jax.experimental.pallas import tpu as pltpu
from jax.experimental.pallas import tpu_sc as plsc

N = 10000
E = 640000
HID = 64
TBL = 128
NC, NS = 2, 16
NW = NC * NS
EW = E // NW
K = 5
EK = E // K
EWK = EK // NW
CG = 400
CS = 2000
BE = 5120
NP = 10240
PW = 3 * NP

_F32 = jnp.float32


def _sc_params():
    cp = pltpu.CompilerParams()
    if "needs_layout_passes" in pltpu.CompilerParams.__dataclass_fields__:
        cp = dataclasses.replace(cp, needs_layout_passes=False)
    return cp


def _node_pre_body(h_ref, loc_ref, wemb_ref, bemb_ref, we1a_ref, we1b_ref,
                   be1_ref, wv_ref, bv_ref, t1_ref, t2_ref, g_ref):
    hh = jnp.dot(h_ref[...], wemb_ref[...], preferred_element_type=_F32) + bemb_ref[...]
    a = jnp.dot(hh, we1a_ref[...], preferred_element_type=_F32) + be1_ref[...]
    b = jnp.dot(hh, we1b_ref[...], preferred_element_type=_F32)
    loc = loc_ref[...]
    pad = jnp.zeros((N, TBL - HID - 2), _F32)
    t1_ref[...] = jnp.concatenate([a, loc, pad], axis=1)
    t2_ref[...] = jnp.concatenate([b, -loc, pad], axis=1)
    g_ref[...] = jnp.dot(hh, wv_ref[...], preferred_element_type=_F32) + bv_ref[...]


def _node_pre(h, loc, W_emb, b_emb, We1a, We1b, be1, Wv, bv):
    return pl.pallas_call(
        _node_pre_body,
        out_shape=[
            jax.ShapeDtypeStruct((N, TBL), _F32),
            jax.ShapeDtypeStruct((N, TBL), _F32),
            jax.ShapeDtypeStruct((N, 1), _F32),
        ],
    )(h, loc, W_emb, b_emb, We1a, We1b, be1, Wv, bv)


def _sc_gather_body(t1_hbm, t2_hbm, rows_hbm, cols_hbm, x_hbm,
                    idx1, idx2, buf1, sem1, sem2):
    wid = lax.axis_index("s") * NC + lax.axis_index("c")
    base0 = wid * EWK

    @pl.loop(0, EWK, step=CG)
    def _(i):
        base = base0 + i
        pltpu.sync_copy(rows_hbm.at[pl.ds(base, CG)], idx1)
        pltpu.sync_copy(cols_hbm.at[pl.ds(base, CG)], idx2)
        pltpu.async_copy(t1_hbm.at[idx1], buf1, sem1).wait()
        pltpu.async_copy(t2_hbm.at[idx2], buf1, sem2, add=True).wait()
        pltpu.sync_copy(buf1, x_hbm.at[pl.ds(base, CG)])


def _sc_gather(T1, T2, rows, cols):
    return pl.kernel(
        _sc_gather_body,
        out_type=jax.ShapeDtypeStruct((EK, TBL), _F32),
        mesh=plsc.VectorSubcoreMesh(core_axis_name="c", subcore_axis_name="s"),
        scratch_types=[
            pltpu.VMEM((CG,), jnp.int32),
            pltpu.VMEM((CG,), jnp.int32),
            pltpu.VMEM((CG, TBL), _F32),
            pltpu.SemaphoreType.DMA,
            pltpu.SemaphoreType.DMA,
        ],
    )(T1, T2, rows, cols)


def _edge_mlp_body(x1_ref, we2_ref, be2_ref, wc1_ref, bc1_ref,
                   wc2t_ref, w2p_ref, tx_ref, ty_ref):
    x = x1_ref[...]
    x64 = x[:, :HID]
    diff = x[:, HID:HID + 2]
    radial = jnp.sum(diff * diff, axis=1, keepdims=True)
    w2sum = w2p_ref[0:1, :] + w2p_ref[1:2, :]
    u = jax.nn.silu(x64 + radial * w2sum)
    m = jax.nn.silu(jnp.dot(u, we2_ref[...], preferred_element_type=_F32) + be2_ref[...])
    p = jax.nn.silu(jnp.dot(m, wc1_ref[...], preferred_element_type=_F32) + bc1_ref[...])
    s = jnp.sum(p * wc2t_ref[...], axis=1, keepdims=True)
    coord = diff / (jnp.sqrt(radial) + 1.0)
    trans = coord * s
    tx_ref[...] = trans[:, 0].reshape(BE // 128, 128)
    ty_ref[...] = trans[:, 1].reshape(BE // 128, 128)


def _edge_mlp(X1, We2, be2, Wc1, bc1, Wc2t, w2p):
    wspec = lambda shp: pl.BlockSpec(shp, lambda i: (0, 0))
    return pl.pallas_call(
        _edge_mlp_body,
        grid=(EK // BE,),
        in_specs=[
            pl.BlockSpec((BE, TBL), lambda i: (i, 0)),
            wspec((HID, HID)),
            wspec((1, HID)),
            wspec((HID, HID)),
            wspec((1, HID)),
            wspec((1, HID)),
            wspec((2, HID)),
        ],
        out_specs=[
            pl.BlockSpec((BE // 128, 128), lambda i: (i, 0)),
            pl.BlockSpec((BE // 128, 128), lambda i: (i, 0)),
        ],
        out_shape=[
            jax.ShapeDtypeStruct((EK // 128, 128), _F32),
            jax.ShapeDtypeStruct((EK // 128, 128), _F32),
        ],
    )(X1, We2, be2, Wc1, bc1, Wc2t, w2p)


def _sc_scatter_body(tx_hbm, ty_hbm, rows_hbm, out_hbm,
                     ibuf, xbuf, ybuf, accx, accy, accc):
    wid = lax.axis_index("s") * NC + lax.axis_index("c")
    base0 = wid * EW
    zf = jnp.zeros((16,), _F32)
    onesf = jnp.full((16,), 1.0, _F32)

    @pl.loop(0, NP, step=16)
    def _(j):
        accx[pl.ds(j, 16)] = zf
        accy[pl.ds(j, 16)] = zf
        accc[pl.ds(j, 16)] = zf

    @pl.loop(0, EW, step=CS)
    def _(i):
        base = base0 + i
        pltpu.sync_copy(rows_hbm.at[pl.ds(base, CS)], ibuf)
        pltpu.sync_copy(tx_hbm.at[pl.ds(base, CS)], xbuf)
        pltpu.sync_copy(ty_hbm.at[pl.ds(base, CS)], ybuf)

        @pl.loop(0, CS, step=16)
        def _(j):
            r16 = ibuf[pl.ds(j, 16)]
            plsc.addupdate_scatter(accx, [r16], xbuf[pl.ds(j, 16)])
            plsc.addupdate_scatter(accy, [r16], ybuf[pl.ds(j, 16)])
            plsc.addupdate_scatter(accc, [r16], onesf)

    obase = wid * PW
    pltpu.sync_copy(accx, out_hbm.at[pl.ds(obase, NP)])
    pltpu.sync_copy(accy, out_hbm.at[pl.ds(obase + NP, NP)])
    pltpu.sync_copy(accc, out_hbm.at[pl.ds(obase + 2 * NP, NP)])


def _sc_scatter(tx, ty, rows):
    return pl.kernel(
        _sc_scatter_body,
        out_type=jax.ShapeDtypeStruct((NW * PW,), _F32),
        mesh=plsc.VectorSubcoreMesh(core_axis_name="c", subcore_axis_name="s"),
        scratch_types=[
            pltpu.VMEM((CS,), jnp.int32),
            pltpu.VMEM((CS,), _F32),
            pltpu.VMEM((CS,), _F32),
            pltpu.VMEM((NP,), _F32),
            pltpu.VMEM((NP,), _F32),
            pltpu.VMEM((NP,), _F32),
        ],
        compiler_params=_sc_params(),
    )(tx, ty, rows)


def _psum_body(part_ref, out_ref):
    out_ref[...] = jnp.sum(part_ref[...], axis=0)


def _psum(partials):
    return pl.pallas_call(
        _psum_body,
        out_shape=jax.ShapeDtypeStruct((3 * (NP // 128), 128), _F32),
    )(partials)


def _final_body(ax_ref, ay_ref, ac_ref, g_ref, vel_ref, loc_ref, h_ref,
                wm1_ref, bm1_ref, wm2_ref, bm2_ref, wm3_ref, bm3_ref,
                ls_ref, mu_ref, lp_ref):
    ax = ax_ref[...]
    ay = ay_ref[...]
    cnt = jnp.maximum(ac_ref[...], 1.0)
    agg = jnp.concatenate([ax / cnt, ay / cnt], axis=1)
    vel = vel_ref[...]
    vel_pred = g_ref[...] * vel + agg
    z = jnp.concatenate([loc_ref[...], vel, h_ref[...]], axis=1)
    z = jax.nn.silu(jnp.dot(z, wm1_ref[...], preferred_element_type=_F32) + bm1_ref[...])
    z = jax.nn.silu(jnp.dot(z, wm2_ref[...], preferred_element_type=_F32) + bm2_ref[...])
    mu = vel_pred + jnp.dot(z, wm3_ref[...], preferred_element_type=_F32) + bm3_ref[...]
    mu_ref[...] = mu
    lp_ref[...] = jnp.zeros(mu.shape, _F32) - ls_ref[...] - 0.5 * jnp.log(2.0 * jnp.pi)


BN = 2000


def _final(ax, ay, ac, g, vel, loc, h, Wm1, bm1, Wm2, bm2, Wm3, bm3, log_std):
    nspec = lambda c: pl.BlockSpec((BN, c), lambda i: (i, 0))
    wspec = lambda shp: pl.BlockSpec(shp, lambda i: (0, 0))
    return pl.pallas_call(
        _final_body,
        grid=(N // BN,),
        in_specs=[
            nspec(1), nspec(1), nspec(1), nspec(1), nspec(2), nspec(2),
            nspec(30),
            wspec((34, 128)), wspec((1, 128)), wspec((128, 128)),
            wspec((1, 128)), wspec((128, 2)), wspec((1, 2)), wspec((1, 2)),
        ],
        out_specs=[
            pl.BlockSpec((BN, 2), lambda i: (i, 0)),
            pl.BlockSpec((BN, 2), lambda i: (i, 0)),
        ],
        out_shape=[
            jax.ShapeDtypeStruct((N, 2), _F32),
            jax.ShapeDtypeStruct((N, 2), _F32),
        ],
    )(ax, ay, ac, g, vel, loc, h, Wm1, bm1, Wm2, bm2, Wm3, bm3, log_std)


def kernel(loc, vel, h, edge_index, W_emb, b_emb, We1, be1, We2, be2, Wc1,
           bc1, Wc2, Wn1, bn1, Wn2, bn2, Wv, bv, Wm1, bm1, Wm2, bm2, Wm3,
           bm3, log_std):
    rows = edge_index[0]
    cols = edge_index[1]

    T1, T2, g = _node_pre(h, loc, W_emb, b_emb.reshape(1, HID),
                          We1[:HID], We1[HID:2 * HID], be1.reshape(1, HID),
                          Wv, bv.reshape(1, 1))
    txs, tys = [], []
    for k in range(K):
        sl = slice(k * EK, (k + 1) * EK)
        X1 = _sc_gather(T1, T2, rows[sl], cols[sl])
        txk, tyk = _edge_mlp(X1, We2, be2.reshape(1, HID), Wc1,
                             bc1.reshape(1, HID), Wc2.reshape(1, HID),
                             We1[2 * HID:2 * HID + 2])
        txs.append(txk)
        tys.append(tyk)
    tx2d = jnp.concatenate(txs, axis=0)
    ty2d = jnp.concatenate(tys, axis=0)
    partials = _sc_scatter(tx2d.reshape(E), ty2d.reshape(E), rows)
    psum = _psum(partials.reshape(NW, 3 * (NP // 128), 128))
    f = psum.reshape(3, NP)
    ax = f[0, :N].reshape(N, 1)
    ay = f[1, :N].reshape(N, 1)
    ac = f[2, :N].reshape(N, 1)
    mu, logp = _final(ax, ay, ac, g, vel,
                      loc, h, Wm1, bm1.reshape(1, 128), Wm2,
                      bm2.reshape(1, 128), Wm3, bm3.reshape(1, 2),
                      log_std.reshape(1, 2))
    return (mu.reshape(100, 100, 2), logp.reshape(100, 100, 2))


# speedup vs baseline: 14.2386x; 1.0013x over previous
import dataclasses

import jax
import jax.numpy as jnp
from jax import lax
from jax.experimental import pallas as pl
from jax.experimental.pallas import tpu as pltpu
from jax.experimental.pallas import tpu_sc as plsc

N = 10000
E = 640000
HID = 64
TBL = 128
NC, NS = 2, 16
NW = NC * NS
EW = E // NW
K = 5
EK = E // K
EWK = EK // NW
CG = 400
CS = 2000
BE = 5120
NP = 10240
PW = 3 * NP

_F32 = jnp.float32


def _sc_params():
    cp = pltpu.CompilerParams()
    if "needs_layout_passes" in pltpu.CompilerParams.__dataclass_fields__:
        cp = dataclasses.replace(cp, needs_layout_passes=False)
    return cp


def _node_pre_body(h_ref, loc_ref, wemb_ref, bemb_ref, we1a_ref, we1b_ref,
                   be1_ref, wv_ref, bv_ref, t1_ref, t2_ref, g_ref):
    hh = jnp.dot(h_ref[...], wemb_ref[...], preferred_element_type=_F32) + bemb_ref[...]
    a = jnp.dot(hh, we1a_ref[...], preferred_element_type=_F32) + be1_ref[...]
    b = jnp.dot(hh, we1b_ref[...], preferred_element_type=_F32)
    loc = loc_ref[...]
    pad = jnp.zeros((N, TBL - HID - 2), _F32)
    t1_ref[...] = jnp.concatenate([a, loc, pad], axis=1)
    t2_ref[...] = jnp.concatenate([b, -loc, pad], axis=1)
    g_ref[...] = jnp.dot(hh, wv_ref[...], preferred_element_type=_F32) + bv_ref[...]


def _node_pre(h, loc, W_emb, b_emb, We1a, We1b, be1, Wv, bv):
    return pl.pallas_call(
        _node_pre_body,
        out_shape=[
            jax.ShapeDtypeStruct((N, TBL), _F32),
            jax.ShapeDtypeStruct((N, TBL), _F32),
            jax.ShapeDtypeStruct((N, 1), _F32),
        ],
    )(h, loc, W_emb, b_emb, We1a, We1b, be1, Wv, bv)


def _sc_gather_body(t1_hbm, t2_hbm, rows_hbm, cols_hbm, x_hbm,
                    i1a, i1b, i2a, i2b, ba, bb,
                    g1a, g1b, g2a, g2b, wsa, wsb):
    wid = lax.axis_index("s") * NC + lax.axis_index("c")
    base0 = wid * EWK
    idx1 = (i1a, i1b)
    idx2 = (i2a, i2b)
    buf = (ba, bb)
    g1s = (g1a, g1b)
    g2s = (g2a, g2b)
    ws = (wsa, wsb)
    nch = EWK // CG

    def g1_desc(p):
        return pltpu.make_async_copy(t1_hbm.at[idx1[p]], buf[p], g1s[p])

    def g2_desc(p):
        return pltpu.make_async_copy(t2_hbm.at[idx2[p]], buf[p], g2s[p])

    def wr_desc(i, p):
        return pltpu.make_async_copy(
            buf[p], x_hbm.at[pl.ds(base0 + i * CG, CG)], ws[p])

    for i in range(nch):
        p = i & 1
        if i >= 2:
            g2_desc(p).wait()
            wr_desc(i - 2, p).start()
            wr_desc(i - 2, p).wait()
        pltpu.sync_copy(rows_hbm.at[pl.ds(base0 + i * CG, CG)], idx1[p])
        pltpu.sync_copy(cols_hbm.at[pl.ds(base0 + i * CG, CG)], idx2[p])
        g1_desc(p).start()
        if i >= 1:
            q = 1 - p
            g1_desc(q).wait()
            g2_desc(q).start(add=True)
    last = nch - 1
    p = last & 1
    q = 1 - p
    g1_desc(p).wait()
    g2_desc(p).start(add=True)
    g2_desc(q).wait()
    wr_desc(last - 1, q).start()
    g2_desc(p).wait()
    wr_desc(last, p).start()
    wr_desc(last - 1, q).wait()
    wr_desc(last, p).wait()


def _sc_gather(T1, T2, rows, cols):
    return pl.kernel(
        _sc_gather_body,
        out_type=jax.ShapeDtypeStruct((EK, TBL), _F32),
        mesh=plsc.VectorSubcoreMesh(core_axis_name="c", subcore_axis_name="s"),
        scratch_types=[
            pltpu.VMEM((CG,), jnp.int32),
            pltpu.VMEM((CG,), jnp.int32),
            pltpu.VMEM((CG,), jnp.int32),
            pltpu.VMEM((CG,), jnp.int32),
            pltpu.VMEM((CG, TBL), _F32),
            pltpu.VMEM((CG, TBL), _F32),
            pltpu.SemaphoreType.DMA,
            pltpu.SemaphoreType.DMA,
            pltpu.SemaphoreType.DMA,
            pltpu.SemaphoreType.DMA,
            pltpu.SemaphoreType.DMA,
            pltpu.SemaphoreType.DMA,
        ],
    )(T1, T2, rows, cols)


def _edge_mlp_body(x1_ref, we2_ref, be2_ref, wc1_ref, bc1_ref,
                   wc2t_ref, w2p_ref, tx_ref, ty_ref):
    x = x1_ref[...]
    x64 = x[:, :HID]
    diff = x[:, HID:HID + 2]
    radial = jnp.sum(diff * diff, axis=1, keepdims=True)
    w2sum = w2p_ref[0:1, :] + w2p_ref[1:2, :]
    u = jax.nn.silu(x64 + radial * w2sum)
    m = jax.nn.silu(jnp.dot(u, we2_ref[...], preferred_element_type=_F32) + be2_ref[...])
    p = jax.nn.silu(jnp.dot(m, wc1_ref[...], preferred_element_type=_F32) + bc1_ref[...])
    s = jnp.sum(p * wc2t_ref[...], axis=1, keepdims=True)
    coord = diff / (jnp.sqrt(radial) + 1.0)
    trans = coord * s
    tx_ref[...] = trans[:, 0].reshape(BE // 128, 128)
    ty_ref[...] = trans[:, 1].reshape(BE // 128, 128)


def _edge_mlp(X1, We2, be2, Wc1, bc1, Wc2t, w2p):
    wspec = lambda shp: pl.BlockSpec(shp, lambda i: (0, 0))
    return pl.pallas_call(
        _edge_mlp_body,
        grid=(EK // BE,),
        in_specs=[
            pl.BlockSpec((BE, TBL), lambda i: (i, 0)),
            wspec((HID, HID)),
            wspec((1, HID)),
            wspec((HID, HID)),
            wspec((1, HID)),
            wspec((1, HID)),
            wspec((2, HID)),
        ],
        out_specs=[
            pl.BlockSpec((BE // 128, 128), lambda i: (i, 0)),
            pl.BlockSpec((BE // 128, 128), lambda i: (i, 0)),
        ],
        out_shape=[
            jax.ShapeDtypeStruct((EK // 128, 128), _F32),
            jax.ShapeDtypeStruct((EK // 128, 128), _F32),
        ],
    )(X1, We2, be2, Wc1, bc1, Wc2t, w2p)


def _sc_scatter_body(tx_hbm, ty_hbm, rows_hbm, out_hbm,
                     ibuf, xbuf, ybuf, accx, accy, accc):
    wid = lax.axis_index("s") * NC + lax.axis_index("c")
    base0 = wid * EW
    zf = jnp.zeros((16,), _F32)
    onesf = jnp.full((16,), 1.0, _F32)

    @pl.loop(0, NP, step=16)
    def _(j):
        accx[pl.ds(j, 16)] = zf
        accy[pl.ds(j, 16)] = zf
        accc[pl.ds(j, 16)] = zf

    @pl.loop(0, EW, step=CS)
    def _(i):
        base = base0 + i
        pltpu.sync_copy(rows_hbm.at[pl.ds(base, CS)], ibuf)
        pltpu.sync_copy(tx_hbm.at[pl.ds(base, CS)], xbuf)
        pltpu.sync_copy(ty_hbm.at[pl.ds(base, CS)], ybuf)

        @pl.loop(0, CS, step=16)
        def _(j):
            r16 = ibuf[pl.ds(j, 16)]
            plsc.addupdate_scatter(accx, [r16], xbuf[pl.ds(j, 16)])
            plsc.addupdate_scatter(accy, [r16], ybuf[pl.ds(j, 16)])
            plsc.addupdate_scatter(accc, [r16], onesf)

    obase = wid * PW
    pltpu.sync_copy(accx, out_hbm.at[pl.ds(obase, NP)])
    pltpu.sync_copy(accy, out_hbm.at[pl.ds(obase + NP, NP)])
    pltpu.sync_copy(accc, out_hbm.at[pl.ds(obase + 2 * NP, NP)])


def _sc_scatter(tx, ty, rows):
    return pl.kernel(
        _sc_scatter_body,
        out_type=jax.ShapeDtypeStruct((NW * PW,), _F32),
        mesh=plsc.VectorSubcoreMesh(core_axis_name="c", subcore_axis_name="s"),
        scratch_types=[
            pltpu.VMEM((CS,), jnp.int32),
            pltpu.VMEM((CS,), _F32),
            pltpu.VMEM((CS,), _F32),
            pltpu.VMEM((NP,), _F32),
            pltpu.VMEM((NP,), _F32),
            pltpu.VMEM((NP,), _F32),
        ],
        compiler_params=_sc_params(),
    )(tx, ty, rows)


def _psum_body(part_ref, out_ref):
    out_ref[...] = jnp.sum(part_ref[...], axis=0)


def _psum(partials):
    return pl.pallas_call(
        _psum_body,
        out_shape=jax.ShapeDtypeStruct((3 * (NP // 128), 128), _F32),
    )(partials)


def _final_body(ax_ref, ay_ref, ac_ref, g_ref, vel_ref, loc_ref, h_ref,
                wm1_ref, bm1_ref, wm2_ref, bm2_ref, wm3_ref, bm3_ref,
                ls_ref, mu_ref, lp_ref):
    ax = ax_ref[...]
    ay = ay_ref[...]
    cnt = jnp.maximum(ac_ref[...], 1.0)
    agg = jnp.concatenate([ax / cnt, ay / cnt], axis=1)
    vel = vel_ref[...]
    vel_pred = g_ref[...] * vel + agg
    z = jnp.concatenate([loc_ref[...], vel, h_ref[...]], axis=1)
    z = jax.nn.silu(jnp.dot(z, wm1_ref[...], preferred_element_type=_F32) + bm1_ref[...])
    z = jax.nn.silu(jnp.dot(z, wm2_ref[...], preferred_element_type=_F32) + bm2_ref[...])
    mu = vel_pred + jnp.dot(z, wm3_ref[...], preferred_element_type=_F32) + bm3_ref[...]
    mu_ref[...] = mu
    lp_ref[...] = jnp.zeros(mu.shape, _F32) - ls_ref[...] - 0.5 * jnp.log(2.0 * jnp.pi)


BN = 2000


def _final(ax, ay, ac, g, vel, loc, h, Wm1, bm1, Wm2, bm2, Wm3, bm3, log_std):
    nspec = lambda c: pl.BlockSpec((BN, c), lambda i: (i, 0))
    wspec = lambda shp: pl.BlockSpec(shp, lambda i: (0, 0))
    return pl.pallas_call(
        _final_body,
        grid=(N // BN,),
        in_specs=[
            nspec(1), nspec(1), nspec(1), nspec(1), nspec(2), nspec(2),
            nspec(30),
            wspec((34, 128)), wspec((1, 128)), wspec((128, 128)),
            wspec((1, 128)), wspec((128, 2)), wspec((1, 2)), wspec((1, 2)),
        ],
        out_specs=[
            pl.BlockSpec((BN, 2), lambda i: (i, 0)),
            pl.BlockSpec((BN, 2), lambda i: (i, 0)),
        ],
        out_shape=[
            jax.ShapeDtypeStruct((N, 2), _F32),
            jax.ShapeDtypeStruct((N, 2), _F32),
        ],
    )(ax, ay, ac, g, vel, loc, h, Wm1, bm1, Wm2, bm2, Wm3, bm3, log_std)


def kernel(loc, vel, h, edge_index, W_emb, b_emb, We1, be1, We2, be2, Wc1,
           bc1, Wc2, Wn1, bn1, Wn2, bn2, Wv, bv, Wm1, bm1, Wm2, bm2, Wm3,
           bm3, log_std):
    rows = edge_index[0]
    cols = edge_index[1]

    T1, T2, g = _node_pre(h, loc, W_emb, b_emb.reshape(1, HID),
                          We1[:HID], We1[HID:2 * HID], be1.reshape(1, HID),
                          Wv, bv.reshape(1, 1))
    txs, tys = [], []
    for k in range(K):
        sl = slice(k * EK, (k + 1) * EK)
        X1 = _sc_gather(T1, T2, rows[sl], cols[sl])
        txk, tyk = _edge_mlp(X1, We2, be2.reshape(1, HID), Wc1,
                             bc1.reshape(1, HID), Wc2.reshape(1, HID),
                             We1[2 * HID:2 * HID + 2])
        txs.append(txk)
        tys.append(tyk)
    tx2d = jnp.concatenate(txs, axis=0)
    ty2d = jnp.concatenate(tys, axis=0)
    partials = _sc_scatter(tx2d.reshape(E), ty2d.reshape(E), rows)
    psum = _psum(partials.reshape(NW, 3 * (NP // 128), 128))
    f = psum.reshape(3, NP)
    ax = f[0, :N].reshape(N, 1)
    ay = f[1, :N].reshape(N, 1)
    ac = f[2, :N].reshape(N, 1)
    mu, logp = _final(ax, ay, ac, g, vel,
                      loc, h, Wm1, bm1.reshape(1, 128), Wm2,
                      bm2.reshape(1, 128), Wm3, bm3.reshape(1, 2),
                      log_std.reshape(1, 2))
    return (mu.reshape(100, 100, 2), logp.reshape(100, 100, 2))


# speedup vs baseline: 14.7515x; 1.0360x over previous
import dataclasses

import jax
import jax.numpy as jnp
from jax import lax
from jax.experimental import pallas as pl
from jax.experimental.pallas import tpu as pltpu
from jax.experimental.pallas import tpu_sc as plsc

N = 10000
E = 640000
HID = 64
TBL = 128
NC, NS = 2, 16
NW = NC * NS
EW = E // NW
K = 5
EK = E // K
EWK = EK // NW
CG = 200
NB = 4
CS = 2000
BE = 5120
NP = 10240
PW = 3 * NP

_F32 = jnp.float32


def _sc_params():
    cp = pltpu.CompilerParams()
    if "needs_layout_passes" in pltpu.CompilerParams.__dataclass_fields__:
        cp = dataclasses.replace(cp, needs_layout_passes=False)
    return cp


def _node_pre_body(h_ref, loc_ref, wemb_ref, bemb_ref, we1a_ref, we1b_ref,
                   be1_ref, wv_ref, bv_ref, t1_ref, t2_ref, g_ref):
    hh = jnp.dot(h_ref[...], wemb_ref[...], preferred_element_type=_F32) + bemb_ref[...]
    a = jnp.dot(hh, we1a_ref[...], preferred_element_type=_F32) + be1_ref[...]
    b = jnp.dot(hh, we1b_ref[...], preferred_element_type=_F32)
    loc = loc_ref[...]
    pad = jnp.zeros((N, TBL - HID - 2), _F32)
    t1_ref[...] = jnp.concatenate([a, loc, pad], axis=1)
    t2_ref[...] = jnp.concatenate([b, -loc, pad], axis=1)
    g_ref[...] = jnp.dot(hh, wv_ref[...], preferred_element_type=_F32) + bv_ref[...]


def _node_pre(h, loc, W_emb, b_emb, We1a, We1b, be1, Wv, bv):
    return pl.pallas_call(
        _node_pre_body,
        out_shape=[
            jax.ShapeDtypeStruct((N, TBL), _F32),
            jax.ShapeDtypeStruct((N, TBL), _F32),
            jax.ShapeDtypeStruct((N, 1), _F32),
        ],
    )(h, loc, W_emb, b_emb, We1a, We1b, be1, Wv, bv)


def _sc_gather_body(t1_hbm, t2_hbm, rows_hbm, cols_hbm, x_hbm, *scr):
    wid = lax.axis_index("s") * NC + lax.axis_index("c")
    base0 = wid * EWK
    idx1 = scr[0:NB]
    idx2 = scr[NB:2 * NB]
    buf = scr[2 * NB:3 * NB]
    g1s = scr[3 * NB:4 * NB]
    g2s = scr[4 * NB:5 * NB]
    ws = scr[5 * NB:6 * NB]
    nch = EWK // CG

    def g1_desc(p):
        return pltpu.make_async_copy(t1_hbm.at[idx1[p]], buf[p], g1s[p])

    def g2_desc(p):
        return pltpu.make_async_copy(t2_hbm.at[idx2[p]], buf[p], g2s[p])

    def wr_desc(i, p):
        return pltpu.make_async_copy(
            buf[p], x_hbm.at[pl.ds(base0 + i * CG, CG)], ws[p])

    for i in range(nch):
        p = i % NB
        if i >= NB:
            wr_desc(i - NB, p).wait()
        pltpu.sync_copy(rows_hbm.at[pl.ds(base0 + i * CG, CG)], idx1[p])
        pltpu.sync_copy(cols_hbm.at[pl.ds(base0 + i * CG, CG)], idx2[p])
        g1_desc(p).start()
        if i >= 1:
            q = (i - 1) % NB
            g1_desc(q).wait()
            g2_desc(q).start(add=True)
        if i >= 2:
            q2 = (i - 2) % NB
            g2_desc(q2).wait()
            wr_desc(i - 2, q2).start()
    for i in (nch - 1,):
        p = i % NB
        g1_desc(p).wait()
        g2_desc(p).start(add=True)
        q2 = (i - 1) % NB
        g2_desc(q2).wait()
        wr_desc(i - 1, q2).start()
        g2_desc(p).wait()
        wr_desc(i, p).start()
        wr_desc(i - 1, q2).wait()
        wr_desc(i, p).wait()
        for j in range(max(nch - NB, 0), nch - 2):
            wr_desc(j, j % NB).wait()


def _sc_gather(T1, T2, rows, cols):
    return pl.kernel(
        _sc_gather_body,
        out_type=jax.ShapeDtypeStruct((EK, TBL), _F32),
        mesh=plsc.VectorSubcoreMesh(core_axis_name="c", subcore_axis_name="s"),
        scratch_types=(
            [pltpu.VMEM((CG,), jnp.int32)] * (2 * NB)
            + [pltpu.VMEM((CG, TBL), _F32)] * NB
            + [pltpu.SemaphoreType.DMA] * (3 * NB)
        ),
    )(T1, T2, rows, cols)


def _edge_mlp_body(x1_ref, we2_ref, be2_ref, wc1_ref, bc1_ref,
                   wc2t_ref, w2p_ref, tx_ref, ty_ref):
    x = x1_ref[...]
    x64 = x[:, :HID]
    diff = x[:, HID:HID + 2]
    radial = jnp.sum(diff * diff, axis=1, keepdims=True)
    w2sum = w2p_ref[0:1, :] + w2p_ref[1:2, :]
    u = jax.nn.silu(x64 + jnp.dot(radial, w2sum, preferred_element_type=_F32))
    m = jax.nn.silu(jnp.dot(u, we2_ref[...], preferred_element_type=_F32) + be2_ref[...])
    p = jax.nn.silu(jnp.dot(m, wc1_ref[...], preferred_element_type=_F32) + bc1_ref[...])
    s = jnp.sum(p * wc2t_ref[...], axis=1, keepdims=True)
    coord = diff / (jnp.sqrt(radial) + 1.0)
    trans = coord * s
    tx_ref[...] = trans[:, 0].reshape(BE // 128, 128)
    ty_ref[...] = trans[:, 1].reshape(BE // 128, 128)


def _edge_mlp(X1, We2, be2, Wc1, bc1, Wc2t, w2p):
    wspec = lambda shp: pl.BlockSpec(shp, lambda i: (0, 0))
    return pl.pallas_call(
        _edge_mlp_body,
        grid=(EK // BE,),
        in_specs=[
            pl.BlockSpec((BE, TBL), lambda i: (i, 0)),
            wspec((HID, HID)),
            wspec((1, HID)),
            wspec((HID, HID)),
            wspec((1, HID)),
            wspec((1, HID)),
            wspec((2, HID)),
        ],
        out_specs=[
            pl.BlockSpec((BE // 128, 128), lambda i: (i, 0)),
            pl.BlockSpec((BE // 128, 128), lambda i: (i, 0)),
        ],
        out_shape=[
            jax.ShapeDtypeStruct((EK // 128, 128), _F32),
            jax.ShapeDtypeStruct((EK // 128, 128), _F32),
        ],
    )(X1, We2, be2, Wc1, bc1, Wc2t, w2p)


def _sc_scatter_body(tx_hbm, ty_hbm, rows_hbm, out_hbm,
                     ibuf, xbuf, ybuf, accx, accy, accc):
    wid = lax.axis_index("s") * NC + lax.axis_index("c")
    base0 = wid * EW
    zf = jnp.zeros((16,), _F32)
    onesf = jnp.full((16,), 1.0, _F32)

    @pl.loop(0, NP, step=16)
    def _(j):
        accx[pl.ds(j, 16)] = zf
        accy[pl.ds(j, 16)] = zf
        accc[pl.ds(j, 16)] = zf

    @pl.loop(0, EW, step=CS)
    def _(i):
        base = base0 + i
        pltpu.sync_copy(rows_hbm.at[pl.ds(base, CS)], ibuf)
        pltpu.sync_copy(tx_hbm.at[pl.ds(base, CS)], xbuf)
        pltpu.sync_copy(ty_hbm.at[pl.ds(base, CS)], ybuf)

        @pl.loop(0, CS, step=16)
        def _(j):
            r16 = ibuf[pl.ds(j, 16)]
            plsc.addupdate_scatter(accx, [r16], xbuf[pl.ds(j, 16)])
            plsc.addupdate_scatter(accy, [r16], ybuf[pl.ds(j, 16)])
            plsc.addupdate_scatter(accc, [r16], onesf)

    obase = wid * PW
    pltpu.sync_copy(accx, out_hbm.at[pl.ds(obase, NP)])
    pltpu.sync_copy(accy, out_hbm.at[pl.ds(obase + NP, NP)])
    pltpu.sync_copy(accc, out_hbm.at[pl.ds(obase + 2 * NP, NP)])


def _sc_scatter(tx, ty, rows):
    return pl.kernel(
        _sc_scatter_body,
        out_type=jax.ShapeDtypeStruct((NW * PW,), _F32),
        mesh=plsc.VectorSubcoreMesh(core_axis_name="c", subcore_axis_name="s"),
        scratch_types=[
            pltpu.VMEM((CS,), jnp.int32),
            pltpu.VMEM((CS,), _F32),
            pltpu.VMEM((CS,), _F32),
            pltpu.VMEM((NP,), _F32),
            pltpu.VMEM((NP,), _F32),
            pltpu.VMEM((NP,), _F32),
        ],
        compiler_params=_sc_params(),
    )(tx, ty, rows)


def _psum_body(part_ref, out_ref):
    out_ref[...] = jnp.sum(part_ref[...], axis=0)


def _psum(partials):
    return pl.pallas_call(
        _psum_body,
        out_shape=jax.ShapeDtypeStruct((3 * (NP // 128), 128), _F32),
    )(partials)


def _final_body(ax_ref, ay_ref, ac_ref, g_ref, vel_ref, loc_ref, h_ref,
                wm1_ref, bm1_ref, wm2_ref, bm2_ref, wm3_ref, bm3_ref,
                ls_ref, mu_ref, lp_ref):
    ax = ax_ref[...]
    ay = ay_ref[...]
    cnt = jnp.maximum(ac_ref[...], 1.0)
    agg = jnp.concatenate([ax / cnt, ay / cnt], axis=1)
    vel = vel_ref[...]
    vel_pred = g_ref[...] * vel + agg
    z = jnp.concatenate([loc_ref[...], vel, h_ref[...]], axis=1)
    z = jax.nn.silu(jnp.dot(z, wm1_ref[...], preferred_element_type=_F32) + bm1_ref[...])
    z = jax.nn.silu(jnp.dot(z, wm2_ref[...], preferred_element_type=_F32) + bm2_ref[...])
    mu = vel_pred + jnp.dot(z, wm3_ref[...], preferred_element_type=_F32) + bm3_ref[...]
    mu_ref[...] = mu
    lp_ref[...] = jnp.zeros(mu.shape, _F32) - ls_ref[...] - 0.5 * jnp.log(2.0 * jnp.pi)


BN = 2000


def _final(ax, ay, ac, g, vel, loc, h, Wm1, bm1, Wm2, bm2, Wm3, bm3, log_std):
    nspec = lambda c: pl.BlockSpec((BN, c), lambda i: (i, 0))
    wspec = lambda shp: pl.BlockSpec(shp, lambda i: (0, 0))
    return pl.pallas_call(
        _final_body,
        grid=(N // BN,),
        in_specs=[
            nspec(1), nspec(1), nspec(1), nspec(1), nspec(2), nspec(2),
            nspec(30),
            wspec((34, 128)), wspec((1, 128)), wspec((128, 128)),
            wspec((1, 128)), wspec((128, 2)), wspec((1, 2)), wspec((1, 2)),
        ],
        out_specs=[
            pl.BlockSpec((BN, 2), lambda i: (i, 0)),
            pl.BlockSpec((BN, 2), lambda i: (i, 0)),
        ],
        out_shape=[
            jax.ShapeDtypeStruct((N, 2), _F32),
            jax.ShapeDtypeStruct((N, 2), _F32),
        ],
    )(ax, ay, ac, g, vel, loc, h, Wm1, bm1, Wm2, bm2, Wm3, bm3, log_std)


def kernel(loc, vel, h, edge_index, W_emb, b_emb, We1, be1, We2, be2, Wc1,
           bc1, Wc2, Wn1, bn1, Wn2, bn2, Wv, bv, Wm1, bm1, Wm2, bm2, Wm3,
           bm3, log_std):
    rows = edge_index[0]
    cols = edge_index[1]

    T1, T2, g = _node_pre(h, loc, W_emb, b_emb.reshape(1, HID),
                          We1[:HID], We1[HID:2 * HID], be1.reshape(1, HID),
                          Wv, bv.reshape(1, 1))
    txs, tys = [], []
    for k in range(K):
        sl = slice(k * EK, (k + 1) * EK)
        X1 = _sc_gather(T1, T2, rows[sl], cols[sl])
        txk, tyk = _edge_mlp(X1, We2, be2.reshape(1, HID), Wc1,
                             bc1.reshape(1, HID), Wc2.reshape(1, HID),
                             We1[2 * HID:2 * HID + 2])
        txs.append(txk)
        tys.append(tyk)
    tx2d = jnp.concatenate(txs, axis=0)
    ty2d = jnp.concatenate(tys, axis=0)
    partials = _sc_scatter(tx2d.reshape(E), ty2d.reshape(E), rows)
    psum = _psum(partials.reshape(NW, 3 * (NP // 128), 128))
    f = psum.reshape(3, NP)
    ax = f[0, :N].reshape(N, 1)
    ay = f[1, :N].reshape(N, 1)
    ac = f[2, :N].reshape(N, 1)
    mu, logp = _final(ax, ay, ac, g, vel,
                      loc, h, Wm1, bm1.reshape(1, 128), Wm2,
                      bm2.reshape(1, 128), Wm3, bm3.reshape(1, 2),
                      log_std.reshape(1, 2))
    return (mu.reshape(100, 100, 2), logp.reshape(100, 100, 2))


# speedup vs baseline: 18.8237x; 1.2761x over previous
import dataclasses

import jax
import jax.numpy as jnp
from jax import lax
from jax.experimental import pallas as pl
from jax.experimental.pallas import tpu as pltpu
from jax.experimental.pallas import tpu_sc as plsc

N = 10000
E = 640000
HID = 64
TBL = 128
NC, NS = 2, 16
NW = NC * NS
EW = E // NW
K = 5
EK = E // K
EWK = EK // NW
CG = 200
NB = 4
CS = 2000
BE = 5120
NP = 10240
PW = 3 * NP

_F32 = jnp.float32


def _sc_params():
    cp = pltpu.CompilerParams()
    if "needs_layout_passes" in pltpu.CompilerParams.__dataclass_fields__:
        cp = dataclasses.replace(cp, needs_layout_passes=False)
    return cp


def _node_pre_body(h_ref, loc_ref, wemb_ref, bemb_ref, we1a_ref, we1b_ref,
                   be1_ref, wv_ref, bv_ref, t1_ref, t2_ref, g_ref):
    hh = jnp.dot(h_ref[...], wemb_ref[...], preferred_element_type=_F32) + bemb_ref[...]
    a = jnp.dot(hh, we1a_ref[...], preferred_element_type=_F32) + be1_ref[...]
    b = jnp.dot(hh, we1b_ref[...], preferred_element_type=_F32)
    loc = loc_ref[...]
    pad = jnp.zeros((N, TBL - HID - 2), _F32)
    t1_ref[...] = jnp.concatenate([a, loc, pad], axis=1)
    t2_ref[...] = jnp.concatenate([b, -loc, pad], axis=1)
    g_ref[...] = jnp.dot(hh, wv_ref[...], preferred_element_type=_F32) + bv_ref[...]


def _node_pre(h, loc, W_emb, b_emb, We1a, We1b, be1, Wv, bv):
    return pl.pallas_call(
        _node_pre_body,
        out_shape=[
            jax.ShapeDtypeStruct((N, TBL), _F32),
            jax.ShapeDtypeStruct((N, TBL), _F32),
            jax.ShapeDtypeStruct((N, 1), _F32),
        ],
    )(h, loc, W_emb, b_emb, We1a, We1b, be1, Wv, bv)


def _sc_gather_body(t1_hbm, t2_hbm, rows_hbm, cols_hbm, x_hbm, *scr):
    wid = lax.axis_index("s") * NC + lax.axis_index("c")
    base0 = wid * EWK
    idx1 = scr[0:NB]
    idx2 = scr[NB:2 * NB]
    buf = scr[2 * NB:3 * NB]
    g1s = scr[3 * NB:4 * NB]
    g2s = scr[4 * NB:5 * NB]
    ws = scr[5 * NB:6 * NB]
    nch = EWK // CG

    def g1_desc(p):
        return pltpu.make_async_copy(t1_hbm.at[idx1[p]], buf[p], g1s[p])

    def g2_desc(p):
        return pltpu.make_async_copy(t2_hbm.at[idx2[p]], buf[p], g2s[p])

    def wr_desc(i, p):
        return pltpu.make_async_copy(
            buf[p], x_hbm.at[pl.ds(base0 + i * CG, CG)], ws[p])

    for i in range(nch):
        p = i % NB
        if i >= NB:
            wr_desc(i - NB, p).wait()
        pltpu.sync_copy(rows_hbm.at[pl.ds(base0 + i * CG, CG)], idx1[p])
        pltpu.sync_copy(cols_hbm.at[pl.ds(base0 + i * CG, CG)], idx2[p])
        g1_desc(p).start()
        if i >= 1:
            q = (i - 1) % NB
            g1_desc(q).wait()
            g2_desc(q).start(add=True)
        if i >= 2:
            q2 = (i - 2) % NB
            g2_desc(q2).wait()
            wr_desc(i - 2, q2).start()
    for i in (nch - 1,):
        p = i % NB
        g1_desc(p).wait()
        g2_desc(p).start(add=True)
        q2 = (i - 1) % NB
        g2_desc(q2).wait()
        wr_desc(i - 1, q2).start()
        g2_desc(p).wait()
        wr_desc(i, p).start()
        wr_desc(i - 1, q2).wait()
        wr_desc(i, p).wait()
        for j in range(max(nch - NB, 0), nch - 2):
            wr_desc(j, j % NB).wait()


def _sc_gather(T1, T2, rows, cols):
    return pl.kernel(
        _sc_gather_body,
        out_type=jax.ShapeDtypeStruct((EK, TBL), _F32),
        mesh=plsc.VectorSubcoreMesh(core_axis_name="c", subcore_axis_name="s"),
        scratch_types=(
            [pltpu.VMEM((CG,), jnp.int32)] * (2 * NB)
            + [pltpu.VMEM((CG, TBL), _F32)] * NB
            + [pltpu.SemaphoreType.DMA] * (3 * NB)
        ),
    )(T1, T2, rows, cols)


def _edge_mlp_body(x1_ref, we2_ref, be2_ref, wc1_ref, bc1_ref,
                   wc2t_ref, w2p_ref, tx_ref, ty_ref):
    x = x1_ref[...]
    x64 = x[:, :HID]
    diff = x[:, HID:HID + 2]
    radial = jnp.sum(diff * diff, axis=1, keepdims=True)
    w2sum = w2p_ref[0:1, :] + w2p_ref[1:2, :]
    u = jax.nn.silu(x64 + jnp.dot(radial, w2sum, preferred_element_type=_F32))
    m = jax.nn.silu(jnp.dot(u, we2_ref[...], preferred_element_type=_F32) + be2_ref[...])
    p = jax.nn.silu(jnp.dot(m, wc1_ref[...], preferred_element_type=_F32) + bc1_ref[...])
    s_t = jax.lax.dot_general(wc2t_ref[...], p, (((1,), (1,)), ((), ())),
                              preferred_element_type=_F32)
    d_t = jnp.transpose(diff)
    dx_t = d_t[0:1, :]
    dy_t = d_t[1:2, :]
    rad_t = dx_t * dx_t + dy_t * dy_t
    ip_t = s_t / (jnp.sqrt(rad_t) + 1.0)
    tx_ref[...] = dx_t * ip_t
    ty_ref[...] = dy_t * ip_t


def _edge_mlp(X1, We2, be2, Wc1, bc1, Wc2t, w2p):
    wspec = lambda shp: pl.BlockSpec(shp, lambda i: (0, 0))
    return pl.pallas_call(
        _edge_mlp_body,
        grid=(EK // BE,),
        in_specs=[
            pl.BlockSpec((BE, TBL), lambda i: (i, 0)),
            wspec((HID, HID)),
            wspec((1, HID)),
            wspec((HID, HID)),
            wspec((1, HID)),
            wspec((1, HID)),
            wspec((2, HID)),
        ],
        out_specs=[
            pl.BlockSpec((1, BE), lambda i: (0, i)),
            pl.BlockSpec((1, BE), lambda i: (0, i)),
        ],
        out_shape=[
            jax.ShapeDtypeStruct((1, EK), _F32),
            jax.ShapeDtypeStruct((1, EK), _F32),
        ],
    )(X1, We2, be2, Wc1, bc1, Wc2t, w2p)


def _sc_scatter_body(tx_hbm, ty_hbm, rows_hbm, out_hbm,
                     ibuf, xbuf, ybuf, accx, accy, accc):
    wid = lax.axis_index("s") * NC + lax.axis_index("c")
    base0 = wid * EW
    zf = jnp.zeros((16,), _F32)
    onesf = jnp.full((16,), 1.0, _F32)

    @pl.loop(0, NP, step=16)
    def _(j):
        accx[pl.ds(j, 16)] = zf
        accy[pl.ds(j, 16)] = zf
        accc[pl.ds(j, 16)] = zf

    @pl.loop(0, EW, step=CS)
    def _(i):
        base = base0 + i
        pltpu.sync_copy(rows_hbm.at[pl.ds(base, CS)], ibuf)
        pltpu.sync_copy(tx_hbm.at[pl.ds(base, CS)], xbuf)
        pltpu.sync_copy(ty_hbm.at[pl.ds(base, CS)], ybuf)

        @pl.loop(0, CS, step=16)
        def _(j):
            r16 = ibuf[pl.ds(j, 16)]
            plsc.addupdate_scatter(accx, [r16], xbuf[pl.ds(j, 16)])
            plsc.addupdate_scatter(accy, [r16], ybuf[pl.ds(j, 16)])
            plsc.addupdate_scatter(accc, [r16], onesf)

    obase = wid * PW
    pltpu.sync_copy(accx, out_hbm.at[pl.ds(obase, NP)])
    pltpu.sync_copy(accy, out_hbm.at[pl.ds(obase + NP, NP)])
    pltpu.sync_copy(accc, out_hbm.at[pl.ds(obase + 2 * NP, NP)])


def _sc_scatter(tx, ty, rows):
    return pl.kernel(
        _sc_scatter_body,
        out_type=jax.ShapeDtypeStruct((NW * PW,), _F32),
        mesh=plsc.VectorSubcoreMesh(core_axis_name="c", subcore_axis_name="s"),
        scratch_types=[
            pltpu.VMEM((CS,), jnp.int32),
            pltpu.VMEM((CS,), _F32),
            pltpu.VMEM((CS,), _F32),
            pltpu.VMEM((NP,), _F32),
            pltpu.VMEM((NP,), _F32),
            pltpu.VMEM((NP,), _F32),
        ],
        compiler_params=_sc_params(),
    )(tx, ty, rows)


def _psum_body(part_ref, out_ref):
    out_ref[...] = jnp.sum(part_ref[...], axis=0)


def _psum(partials):
    return pl.pallas_call(
        _psum_body,
        out_shape=jax.ShapeDtypeStruct((3 * (NP // 128), 128), _F32),
    )(partials)


def _final_body(ax_ref, ay_ref, ac_ref, g_ref, vel_ref, loc_ref, h_ref,
                wm1_ref, bm1_ref, wm2_ref, bm2_ref, wm3_ref, bm3_ref,
                ls_ref, mu_ref, lp_ref):
    ax = ax_ref[...]
    ay = ay_ref[...]
    cnt = jnp.maximum(ac_ref[...], 1.0)
    agg = jnp.concatenate([ax / cnt, ay / cnt], axis=1)
    vel = vel_ref[...]
    vel_pred = g_ref[...] * vel + agg
    z = jnp.concatenate([loc_ref[...], vel, h_ref[...]], axis=1)
    z = jax.nn.silu(jnp.dot(z, wm1_ref[...], preferred_element_type=_F32) + bm1_ref[...])
    z = jax.nn.silu(jnp.dot(z, wm2_ref[...], preferred_element_type=_F32) + bm2_ref[...])
    mu = vel_pred + jnp.dot(z, wm3_ref[...], preferred_element_type=_F32) + bm3_ref[...]
    mu_ref[...] = mu
    lp_ref[...] = jnp.zeros(mu.shape, _F32) - ls_ref[...] - 0.5 * jnp.log(2.0 * jnp.pi)


BN = 2000


def _final(ax, ay, ac, g, vel, loc, h, Wm1, bm1, Wm2, bm2, Wm3, bm3, log_std):
    nspec = lambda c: pl.BlockSpec((BN, c), lambda i: (i, 0))
    wspec = lambda shp: pl.BlockSpec(shp, lambda i: (0, 0))
    return pl.pallas_call(
        _final_body,
        grid=(N // BN,),
        in_specs=[
            nspec(1), nspec(1), nspec(1), nspec(1), nspec(2), nspec(2),
            nspec(30),
            wspec((34, 128)), wspec((1, 128)), wspec((128, 128)),
            wspec((1, 128)), wspec((128, 2)), wspec((1, 2)), wspec((1, 2)),
        ],
        out_specs=[
            pl.BlockSpec((BN, 2), lambda i: (i, 0)),
            pl.BlockSpec((BN, 2), lambda i: (i, 0)),
        ],
        out_shape=[
            jax.ShapeDtypeStruct((N, 2), _F32),
            jax.ShapeDtypeStruct((N, 2), _F32),
        ],
    )(ax, ay, ac, g, vel, loc, h, Wm1, bm1, Wm2, bm2, Wm3, bm3, log_std)


def kernel(loc, vel, h, edge_index, W_emb, b_emb, We1, be1, We2, be2, Wc1,
           bc1, Wc2, Wn1, bn1, Wn2, bn2, Wv, bv, Wm1, bm1, Wm2, bm2, Wm3,
           bm3, log_std):
    rows = edge_index[0]
    cols = edge_index[1]

    T1, T2, g = _node_pre(h, loc, W_emb, b_emb.reshape(1, HID),
                          We1[:HID], We1[HID:2 * HID], be1.reshape(1, HID),
                          Wv, bv.reshape(1, 1))
    txs, tys = [], []
    for k in range(K):
        sl = slice(k * EK, (k + 1) * EK)
        X1 = _sc_gather(T1, T2, rows[sl], cols[sl])
        txk, tyk = _edge_mlp(X1, We2, be2.reshape(1, HID), Wc1,
                             bc1.reshape(1, HID), Wc2.reshape(1, HID),
                             We1[2 * HID:2 * HID + 2])
        txs.append(txk.reshape(EK))
        tys.append(tyk.reshape(EK))
    tx1 = jnp.concatenate(txs, axis=0)
    ty1 = jnp.concatenate(tys, axis=0)
    partials = _sc_scatter(tx1, ty1, rows)
    psum = _psum(partials.reshape(NW, 3 * (NP // 128), 128))
    f = psum.reshape(3, NP)
    ax = f[0, :N].reshape(N, 1)
    ay = f[1, :N].reshape(N, 1)
    ac = f[2, :N].reshape(N, 1)
    mu, logp = _final(ax, ay, ac, g, vel,
                      loc, h, Wm1, bm1.reshape(1, 128), Wm2,
                      bm2.reshape(1, 128), Wm3, bm3.reshape(1, 2),
                      log_std.reshape(1, 2))
    return (mu.reshape(100, 100, 2), logp.reshape(100, 100, 2))


# speedup vs baseline: 19.3223x; 1.0265x over previous
import dataclasses

import jax
import jax.numpy as jnp
from jax import lax
from jax.experimental import pallas as pl
from jax.experimental.pallas import tpu as pltpu
from jax.experimental.pallas import tpu_sc as plsc

N = 10000
E = 640000
HID = 64
TBL = 128
NC, NS = 2, 16
NW = NC * NS
EW = E // NW
K = 5
EK = E // K
EWK = EK // NW
CG = 200
NB = 4
CS = 2000
BE = 5120
NP = 10240
PW = 3 * NP

_F32 = jnp.float32


def _sc_params():
    cp = pltpu.CompilerParams()
    if "needs_layout_passes" in pltpu.CompilerParams.__dataclass_fields__:
        cp = dataclasses.replace(cp, needs_layout_passes=False)
    return cp


def _node_pre_body(h_ref, loc_ref, wemb_ref, bemb_ref, we1a_ref, we1b_ref,
                   be1_ref, wv_ref, bv_ref, t1_ref, t2_ref, g_ref):
    hh = jnp.dot(h_ref[...], wemb_ref[...], preferred_element_type=_F32) + bemb_ref[...]
    a = jnp.dot(hh, we1a_ref[...], preferred_element_type=_F32) + be1_ref[...]
    b = jnp.dot(hh, we1b_ref[...], preferred_element_type=_F32)
    loc = loc_ref[...]
    pad = jnp.zeros((N, TBL - HID - 2), _F32)
    t1_ref[...] = jnp.concatenate([a, loc, pad], axis=1)
    t2_ref[...] = jnp.concatenate([b, -loc, pad], axis=1)
    g_ref[...] = jnp.dot(hh, wv_ref[...], preferred_element_type=_F32) + bv_ref[...]


def _node_pre(h, loc, W_emb, b_emb, We1a, We1b, be1, Wv, bv):
    return pl.pallas_call(
        _node_pre_body,
        out_shape=[
            jax.ShapeDtypeStruct((N, TBL), _F32),
            jax.ShapeDtypeStruct((N, TBL), _F32),
            jax.ShapeDtypeStruct((N, 1), _F32),
        ],
    )(h, loc, W_emb, b_emb, We1a, We1b, be1, Wv, bv)


def _sc_gather_body(t1_hbm, t2_hbm, rows_hbm, cols_hbm, x_hbm, *scr):
    wid = lax.axis_index("s") * NC + lax.axis_index("c")
    base0 = wid * EWK
    idx1 = scr[0:NB]
    idx2 = scr[NB:2 * NB]
    buf = scr[2 * NB:3 * NB]
    g1s = scr[3 * NB:4 * NB]
    g2s = scr[4 * NB:5 * NB]
    ws = scr[5 * NB:6 * NB]
    nch = EWK // CG

    def g1_desc(p):
        return pltpu.make_async_copy(t1_hbm.at[idx1[p]], buf[p], g1s[p])

    def g2_desc(p):
        return pltpu.make_async_copy(t2_hbm.at[idx2[p]], buf[p], g2s[p])

    def wr_desc(i, p):
        return pltpu.make_async_copy(
            buf[p], x_hbm.at[pl.ds(base0 + i * CG, CG)], ws[p])

    for i in range(nch):
        p = i % NB
        if i >= NB:
            wr_desc(i - NB, p).wait()
        pltpu.sync_copy(rows_hbm.at[pl.ds(base0 + i * CG, CG)], idx1[p])
        pltpu.sync_copy(cols_hbm.at[pl.ds(base0 + i * CG, CG)], idx2[p])
        g1_desc(p).start()
        if i >= 1:
            q = (i - 1) % NB
            g1_desc(q).wait()
            g2_desc(q).start(add=True)
        if i >= 2:
            q2 = (i - 2) % NB
            g2_desc(q2).wait()
            wr_desc(i - 2, q2).start()
    for i in (nch - 1,):
        p = i % NB
        g1_desc(p).wait()
        g2_desc(p).start(add=True)
        q2 = (i - 1) % NB
        g2_desc(q2).wait()
        wr_desc(i - 1, q2).start()
        g2_desc(p).wait()
        wr_desc(i, p).start()
        wr_desc(i - 1, q2).wait()
        wr_desc(i, p).wait()
        for j in range(max(nch - NB, 0), nch - 2):
            wr_desc(j, j % NB).wait()


def _sc_gather(T1, T2, rows, cols):
    return pl.kernel(
        _sc_gather_body,
        out_type=jax.ShapeDtypeStruct((EK, TBL), _F32),
        mesh=plsc.VectorSubcoreMesh(core_axis_name="c", subcore_axis_name="s"),
        scratch_types=(
            [pltpu.VMEM((CG,), jnp.int32)] * (2 * NB)
            + [pltpu.VMEM((CG, TBL), _F32)] * NB
            + [pltpu.SemaphoreType.DMA] * (3 * NB)
        ),
    )(T1, T2, rows, cols)


def _edge_mlp_body(x1_ref, we2_ref, be2_ref, wc1_ref, bc1_ref,
                   wc2t_ref, w2p_ref, tx_ref, ty_ref):
    x = x1_ref[...]
    x64 = x[:, :HID]
    diff = x[:, HID:HID + 2]
    u = jax.nn.silu(x64 + jnp.dot(diff * diff, w2p_ref[...],
                                  preferred_element_type=_F32))
    m = jax.nn.silu(jnp.dot(u, we2_ref[...], preferred_element_type=_F32) + be2_ref[...])
    p = jax.nn.silu(jnp.dot(m, wc1_ref[...], preferred_element_type=_F32) + bc1_ref[...])
    s_t = jax.lax.dot_general(wc2t_ref[...], p, (((1,), (1,)), ((), ())),
                              preferred_element_type=_F32)
    d_t = jnp.transpose(diff)
    dx_t = d_t[0:1, :]
    dy_t = d_t[1:2, :]
    rad_t = dx_t * dx_t + dy_t * dy_t
    ip_t = s_t / (jnp.sqrt(rad_t) + 1.0)
    tx_ref[...] = dx_t * ip_t
    ty_ref[...] = dy_t * ip_t


def _edge_mlp(X1, We2, be2, Wc1, bc1, Wc2t, w2p):
    wspec = lambda shp: pl.BlockSpec(shp, lambda i: (0, 0))
    return pl.pallas_call(
        _edge_mlp_body,
        grid=(EK // BE,),
        in_specs=[
            pl.BlockSpec((BE, TBL), lambda i: (i, 0)),
            wspec((HID, HID)),
            wspec((1, HID)),
            wspec((HID, HID)),
            wspec((1, HID)),
            wspec((1, HID)),
            wspec((2, HID)),
        ],
        out_specs=[
            pl.BlockSpec((1, BE), lambda i: (0, i)),
            pl.BlockSpec((1, BE), lambda i: (0, i)),
        ],
        out_shape=[
            jax.ShapeDtypeStruct((1, EK), _F32),
            jax.ShapeDtypeStruct((1, EK), _F32),
        ],
    )(X1, We2, be2, Wc1, bc1, Wc2t, w2p)


def _sc_scatter_body(tx_hbm, ty_hbm, rows_hbm, out_hbm,
                     ibuf, xbuf, ybuf, accx, accy, accc):
    wid = lax.axis_index("s") * NC + lax.axis_index("c")
    base0 = wid * EW
    zf = jnp.zeros((16,), _F32)
    onesf = jnp.full((16,), 1.0, _F32)

    @pl.loop(0, NP, step=16)
    def _(j):
        accx[pl.ds(j, 16)] = zf
        accy[pl.ds(j, 16)] = zf
        accc[pl.ds(j, 16)] = zf

    @pl.loop(0, EW, step=CS)
    def _(i):
        base = base0 + i
        pltpu.sync_copy(rows_hbm.at[pl.ds(base, CS)], ibuf)
        pltpu.sync_copy(tx_hbm.at[pl.ds(base, CS)], xbuf)
        pltpu.sync_copy(ty_hbm.at[pl.ds(base, CS)], ybuf)

        @pl.loop(0, CS, step=16)
        def _(j):
            r16 = ibuf[pl.ds(j, 16)]
            plsc.addupdate_scatter(accx, [r16], xbuf[pl.ds(j, 16)])
            plsc.addupdate_scatter(accy, [r16], ybuf[pl.ds(j, 16)])
            plsc.addupdate_scatter(accc, [r16], onesf)

    obase = wid * PW
    pltpu.sync_copy(accx, out_hbm.at[pl.ds(obase, NP)])
    pltpu.sync_copy(accy, out_hbm.at[pl.ds(obase + NP, NP)])
    pltpu.sync_copy(accc, out_hbm.at[pl.ds(obase + 2 * NP, NP)])


def _sc_scatter(tx, ty, rows):
    return pl.kernel(
        _sc_scatter_body,
        out_type=jax.ShapeDtypeStruct((NW * PW,), _F32),
        mesh=plsc.VectorSubcoreMesh(core_axis_name="c", subcore_axis_name="s"),
        scratch_types=[
            pltpu.VMEM((CS,), jnp.int32),
            pltpu.VMEM((CS,), _F32),
            pltpu.VMEM((CS,), _F32),
            pltpu.VMEM((NP,), _F32),
            pltpu.VMEM((NP,), _F32),
            pltpu.VMEM((NP,), _F32),
        ],
        compiler_params=_sc_params(),
    )(tx, ty, rows)


def _psum_body(part_ref, out_ref):
    out_ref[...] = jnp.sum(part_ref[...], axis=0)


def _psum(partials):
    return pl.pallas_call(
        _psum_body,
        out_shape=jax.ShapeDtypeStruct((3 * (NP // 128), 128), _F32),
    )(partials)


def _final_body(ax_ref, ay_ref, ac_ref, g_ref, vel_ref, loc_ref, h_ref,
                wm1_ref, bm1_ref, wm2_ref, bm2_ref, wm3_ref, bm3_ref,
                ls_ref, mu_ref, lp_ref):
    ax = ax_ref[...]
    ay = ay_ref[...]
    cnt = jnp.maximum(ac_ref[...], 1.0)
    agg = jnp.concatenate([ax / cnt, ay / cnt], axis=1)
    vel = vel_ref[...]
    vel_pred = g_ref[...] * vel + agg
    z = jnp.concatenate([loc_ref[...], vel, h_ref[...]], axis=1)
    z = jax.nn.silu(jnp.dot(z, wm1_ref[...], preferred_element_type=_F32) + bm1_ref[...])
    z = jax.nn.silu(jnp.dot(z, wm2_ref[...], preferred_element_type=_F32) + bm2_ref[...])
    mu = vel_pred + jnp.dot(z, wm3_ref[...], preferred_element_type=_F32) + bm3_ref[...]
    mu_ref[...] = mu
    lp_ref[...] = jnp.zeros(mu.shape, _F32) - ls_ref[...] - 0.5 * jnp.log(2.0 * jnp.pi)


BN = 2000


def _final(ax, ay, ac, g, vel, loc, h, Wm1, bm1, Wm2, bm2, Wm3, bm3, log_std):
    nspec = lambda c: pl.BlockSpec((BN, c), lambda i: (i, 0))
    wspec = lambda shp: pl.BlockSpec(shp, lambda i: (0, 0))
    return pl.pallas_call(
        _final_body,
        grid=(N // BN,),
        in_specs=[
            nspec(1), nspec(1), nspec(1), nspec(1), nspec(2), nspec(2),
            nspec(30),
            wspec((34, 128)), wspec((1, 128)), wspec((128, 128)),
            wspec((1, 128)), wspec((128, 2)), wspec((1, 2)), wspec((1, 2)),
        ],
        out_specs=[
            pl.BlockSpec((BN, 2), lambda i: (i, 0)),
            pl.BlockSpec((BN, 2), lambda i: (i, 0)),
        ],
        out_shape=[
            jax.ShapeDtypeStruct((N, 2), _F32),
            jax.ShapeDtypeStruct((N, 2), _F32),
        ],
    )(ax, ay, ac, g, vel, loc, h, Wm1, bm1, Wm2, bm2, Wm3, bm3, log_std)


def kernel(loc, vel, h, edge_index, W_emb, b_emb, We1, be1, We2, be2, Wc1,
           bc1, Wc2, Wn1, bn1, Wn2, bn2, Wv, bv, Wm1, bm1, Wm2, bm2, Wm3,
           bm3, log_std):
    rows = edge_index[0]
    cols = edge_index[1]

    T1, T2, g = _node_pre(h, loc, W_emb, b_emb.reshape(1, HID),
                          We1[:HID], We1[HID:2 * HID], be1.reshape(1, HID),
                          Wv, bv.reshape(1, 1))
    w2sum = We1[2 * HID:2 * HID + 1] + We1[2 * HID + 1:2 * HID + 2]
    w2s2 = jnp.concatenate([w2sum, w2sum], axis=0)
    txs, tys = [], []
    for k in range(K):
        sl = slice(k * EK, (k + 1) * EK)
        X1 = _sc_gather(T1, T2, rows[sl], cols[sl])
        txk, tyk = _edge_mlp(X1, We2, be2.reshape(1, HID), Wc1,
                             bc1.reshape(1, HID), Wc2.reshape(1, HID), w2s2)
        txs.append(txk.reshape(EK))
        tys.append(tyk.reshape(EK))
    tx1 = jnp.concatenate(txs, axis=0)
    ty1 = jnp.concatenate(tys, axis=0)
    partials = _sc_scatter(tx1, ty1, rows)
    psum = _psum(partials.reshape(NW, 3 * (NP // 128), 128))
    f = psum.reshape(3, NP)
    ax = f[0, :N].reshape(N, 1)
    ay = f[1, :N].reshape(N, 1)
    ac = f[2, :N].reshape(N, 1)
    mu, logp = _final(ax, ay, ac, g, vel,
                      loc, h, Wm1, bm1.reshape(1, 128), Wm2,
                      bm2.reshape(1, 128), Wm3, bm3.reshape(1, 2),
                      log_std.reshape(1, 2))
    return (mu.reshape(100, 100, 2), logp.reshape(100, 100, 2))


# speedup vs baseline: 19.4122x; 1.0046x over previous
import dataclasses

import jax
import jax.numpy as jnp
from jax import lax
from jax.experimental import pallas as pl
from jax.experimental.pallas import tpu as pltpu
from jax.experimental.pallas import tpu_sc as plsc

N = 10000
E = 640000
HID = 64
TBL = 128
NC, NS = 2, 16
NW = NC * NS
EW = E // NW
K = 5
EK = E // K
EWK = EK // NW
CG = 200
NB = 4
CS = 2000
EH = E // 2
EWH = EH // NW
BE = 5120
NP = 10240
PW = 3 * NP

_F32 = jnp.float32


def _sc_params():
    cp = pltpu.CompilerParams()
    if "needs_layout_passes" in pltpu.CompilerParams.__dataclass_fields__:
        cp = dataclasses.replace(cp, needs_layout_passes=False)
    return cp


def _node_pre_body(h_ref, loc_ref, wemb_ref, bemb_ref, we1a_ref, we1b_ref,
                   be1_ref, wv_ref, bv_ref, t1_ref, t2_ref, g_ref):
    hh = jnp.dot(h_ref[...], wemb_ref[...], preferred_element_type=_F32) + bemb_ref[...]
    a = jnp.dot(hh, we1a_ref[...], preferred_element_type=_F32) + be1_ref[...]
    b = jnp.dot(hh, we1b_ref[...], preferred_element_type=_F32)
    loc = loc_ref[...]
    pad = jnp.zeros((N, TBL - HID - 2), _F32)
    t1_ref[...] = jnp.concatenate([a, loc, pad], axis=1)
    t2_ref[...] = jnp.concatenate([b, -loc, pad], axis=1)
    g_ref[...] = jnp.dot(hh, wv_ref[...], preferred_element_type=_F32) + bv_ref[...]


def _node_pre(h, loc, W_emb, b_emb, We1a, We1b, be1, Wv, bv):
    return pl.pallas_call(
        _node_pre_body,
        out_shape=[
            jax.ShapeDtypeStruct((N, TBL), _F32),
            jax.ShapeDtypeStruct((N, TBL), _F32),
            jax.ShapeDtypeStruct((N, 1), _F32),
        ],
    )(h, loc, W_emb, b_emb, We1a, We1b, be1, Wv, bv)


def _sc_gather_body(t1_hbm, t2_hbm, rows_hbm, cols_hbm, x_hbm, *scr):
    wid = lax.axis_index("s") * NC + lax.axis_index("c")
    base0 = wid * EWK
    idx1 = scr[0:NB]
    idx2 = scr[NB:2 * NB]
    buf = scr[2 * NB:3 * NB]
    g1s = scr[3 * NB:4 * NB]
    g2s = scr[4 * NB:5 * NB]
    ws = scr[5 * NB:6 * NB]
    nch = EWK // CG

    def g1_desc(p):
        return pltpu.make_async_copy(t1_hbm.at[idx1[p]], buf[p], g1s[p])

    def g2_desc(p):
        return pltpu.make_async_copy(t2_hbm.at[idx2[p]], buf[p], g2s[p])

    def wr_desc(i, p):
        return pltpu.make_async_copy(
            buf[p], x_hbm.at[pl.ds(base0 + i * CG, CG)], ws[p])

    for i in range(nch):
        p = i % NB
        if i >= NB:
            wr_desc(i - NB, p).wait()
        pltpu.sync_copy(rows_hbm.at[pl.ds(base0 + i * CG, CG)], idx1[p])
        pltpu.sync_copy(cols_hbm.at[pl.ds(base0 + i * CG, CG)], idx2[p])
        g1_desc(p).start()
        if i >= 1:
            q = (i - 1) % NB
            g1_desc(q).wait()
            g2_desc(q).start(add=True)
        if i >= 2:
            q2 = (i - 2) % NB
            g2_desc(q2).wait()
            wr_desc(i - 2, q2).start()
    for i in (nch - 1,):
        p = i % NB
        g1_desc(p).wait()
        g2_desc(p).start(add=True)
        q2 = (i - 1) % NB
        g2_desc(q2).wait()
        wr_desc(i - 1, q2).start()
        g2_desc(p).wait()
        wr_desc(i, p).start()
        wr_desc(i - 1, q2).wait()
        wr_desc(i, p).wait()
        for j in range(max(nch - NB, 0), nch - 2):
            wr_desc(j, j % NB).wait()


def _sc_gather(T1, T2, rows, cols):
    return pl.kernel(
        _sc_gather_body,
        out_type=jax.ShapeDtypeStruct((EK, TBL), _F32),
        mesh=plsc.VectorSubcoreMesh(core_axis_name="c", subcore_axis_name="s"),
        scratch_types=(
            [pltpu.VMEM((CG,), jnp.int32)] * (2 * NB)
            + [pltpu.VMEM((CG, TBL), _F32)] * NB
            + [pltpu.SemaphoreType.DMA] * (3 * NB)
        ),
    )(T1, T2, rows, cols)


def _edge_mlp_body(x1_ref, we2_ref, be2_ref, wc1_ref, bc1_ref,
                   wc2t_ref, w2p_ref, tx_ref, ty_ref):
    x = x1_ref[...]
    x64 = x[:, :HID]
    diff = x[:, HID:HID + 2]
    u = jax.nn.silu(x64 + jnp.dot(diff * diff, w2p_ref[...],
                                  preferred_element_type=_F32))
    m = jax.nn.silu(jnp.dot(u, we2_ref[...], preferred_element_type=_F32) + be2_ref[...])
    p = jax.nn.silu(jnp.dot(m, wc1_ref[...], preferred_element_type=_F32) + bc1_ref[...])
    s_t = jax.lax.dot_general(wc2t_ref[...], p, (((1,), (1,)), ((), ())),
                              preferred_element_type=_F32)
    d_t = jnp.transpose(diff)
    dx_t = d_t[0:1, :]
    dy_t = d_t[1:2, :]
    rad_t = dx_t * dx_t + dy_t * dy_t
    ip_t = s_t / (jnp.sqrt(rad_t) + 1.0)
    tx_ref[...] = dx_t * ip_t
    ty_ref[...] = dy_t * ip_t


def _edge_mlp(X1, We2, be2, Wc1, bc1, Wc2t, w2p):
    wspec = lambda shp: pl.BlockSpec(shp, lambda i: (0, 0))
    return pl.pallas_call(
        _edge_mlp_body,
        grid=(EK // BE,),
        in_specs=[
            pl.BlockSpec((BE, TBL), lambda i: (i, 0)),
            wspec((HID, HID)),
            wspec((1, HID)),
            wspec((HID, HID)),
            wspec((1, HID)),
            wspec((1, HID)),
            wspec((2, HID)),
        ],
        out_specs=[
            pl.BlockSpec((1, BE), lambda i: (0, i)),
            pl.BlockSpec((1, BE), lambda i: (0, i)),
        ],
        out_shape=[
            jax.ShapeDtypeStruct((1, EK), _F32),
            jax.ShapeDtypeStruct((1, EK), _F32),
        ],
    )(X1, We2, be2, Wc1, bc1, Wc2t, w2p)


def _sc_scatter_body(tx_hbm, ty_hbm, rows_hbm, out_hbm,
                     ibuf, xbuf, ybuf, accx, accy, accc):
    wid = lax.axis_index("s") * NC + lax.axis_index("c")
    base0 = wid * EWH
    zf = jnp.zeros((16,), _F32)
    onesf = jnp.full((16,), 1.0, _F32)

    @pl.loop(0, NP, step=16)
    def _(j):
        accx[pl.ds(j, 16)] = zf
        accy[pl.ds(j, 16)] = zf
        accc[pl.ds(j, 16)] = zf

    @pl.loop(0, EWH, step=CS)
    def _(i):
        base = base0 + i
        pltpu.sync_copy(rows_hbm.at[pl.ds(base, CS)], ibuf)
        pltpu.sync_copy(tx_hbm.at[pl.ds(base, CS)], xbuf)
        pltpu.sync_copy(ty_hbm.at[pl.ds(base, CS)], ybuf)

        @pl.loop(0, CS, step=16)
        def _(j):
            r16 = ibuf[pl.ds(j, 16)]
            plsc.addupdate_scatter(accx, [r16], xbuf[pl.ds(j, 16)])
            plsc.addupdate_scatter(accy, [r16], ybuf[pl.ds(j, 16)])
            plsc.addupdate_scatter(accc, [r16], onesf)

    obase = wid * PW
    pltpu.sync_copy(accx, out_hbm.at[pl.ds(obase, NP)])
    pltpu.sync_copy(accy, out_hbm.at[pl.ds(obase + NP, NP)])
    pltpu.sync_copy(accc, out_hbm.at[pl.ds(obase + 2 * NP, NP)])


def _sc_scatter(tx, ty, rows):
    return pl.kernel(
        _sc_scatter_body,
        out_type=jax.ShapeDtypeStruct((NW * PW,), _F32),
        mesh=plsc.VectorSubcoreMesh(core_axis_name="c", subcore_axis_name="s"),
        scratch_types=[
            pltpu.VMEM((CS,), jnp.int32),
            pltpu.VMEM((CS,), _F32),
            pltpu.VMEM((CS,), _F32),
            pltpu.VMEM((NP,), _F32),
            pltpu.VMEM((NP,), _F32),
            pltpu.VMEM((NP,), _F32),
        ],
        compiler_params=_sc_params(),
    )(tx, ty, rows)


def _psum_body(part_ref, out_ref):
    out_ref[...] = jnp.sum(part_ref[...], axis=0)


def _psum(partials):
    return pl.pallas_call(
        _psum_body,
        out_shape=jax.ShapeDtypeStruct((3 * (NP // 128), 128), _F32),
    )(partials)


def _final_body(ax_ref, ay_ref, ac_ref, g_ref, vel_ref, loc_ref, h_ref,
                wm1_ref, bm1_ref, wm2_ref, bm2_ref, wm3_ref, bm3_ref,
                ls_ref, mu_ref, lp_ref):
    ax = ax_ref[...]
    ay = ay_ref[...]
    cnt = jnp.maximum(ac_ref[...], 1.0)
    agg = jnp.concatenate([ax / cnt, ay / cnt], axis=1)
    vel = vel_ref[...]
    vel_pred = g_ref[...] * vel + agg
    z = jnp.concatenate([loc_ref[...], vel, h_ref[...]], axis=1)
    z = jax.nn.silu(jnp.dot(z, wm1_ref[...], preferred_element_type=_F32) + bm1_ref[...])
    z = jax.nn.silu(jnp.dot(z, wm2_ref[...], preferred_element_type=_F32) + bm2_ref[...])
    mu = vel_pred + jnp.dot(z, wm3_ref[...], preferred_element_type=_F32) + bm3_ref[...]
    mu_ref[...] = mu
    lp_ref[...] = jnp.zeros(mu.shape, _F32) - ls_ref[...] - 0.5 * jnp.log(2.0 * jnp.pi)


BN = 2000


def _final(ax, ay, ac, g, vel, loc, h, Wm1, bm1, Wm2, bm2, Wm3, bm3, log_std):
    nspec = lambda c: pl.BlockSpec((BN, c), lambda i: (i, 0))
    wspec = lambda shp: pl.BlockSpec(shp, lambda i: (0, 0))
    return pl.pallas_call(
        _final_body,
        grid=(N // BN,),
        in_specs=[
            nspec(1), nspec(1), nspec(1), nspec(1), nspec(2), nspec(2),
            nspec(30),
            wspec((34, 128)), wspec((1, 128)), wspec((128, 128)),
            wspec((1, 128)), wspec((128, 2)), wspec((1, 2)), wspec((1, 2)),
        ],
        out_specs=[
            pl.BlockSpec((BN, 2), lambda i: (i, 0)),
            pl.BlockSpec((BN, 2), lambda i: (i, 0)),
        ],
        out_shape=[
            jax.ShapeDtypeStruct((N, 2), _F32),
            jax.ShapeDtypeStruct((N, 2), _F32),
        ],
    )(ax, ay, ac, g, vel, loc, h, Wm1, bm1, Wm2, bm2, Wm3, bm3, log_std)


def kernel(loc, vel, h, edge_index, W_emb, b_emb, We1, be1, We2, be2, Wc1,
           bc1, Wc2, Wn1, bn1, Wn2, bn2, Wv, bv, Wm1, bm1, Wm2, bm2, Wm3,
           bm3, log_std):
    rows = edge_index[0]
    cols = edge_index[1]

    T1, T2, g = _node_pre(h, loc, W_emb, b_emb.reshape(1, HID),
                          We1[:HID], We1[HID:2 * HID], be1.reshape(1, HID),
                          Wv, bv.reshape(1, 1))
    w2sum = We1[2 * HID:2 * HID + 1] + We1[2 * HID + 1:2 * HID + 2]
    w2s2 = jnp.concatenate([w2sum, w2sum], axis=0)
    txs, tys = [], []
    for k in range(K):
        sl = slice(k * EK, (k + 1) * EK)
        X1 = _sc_gather(T1, T2, rows[sl], cols[sl])
        txk, tyk = _edge_mlp(X1, We2, be2.reshape(1, HID), Wc1,
                             bc1.reshape(1, HID), Wc2.reshape(1, HID), w2s2)
        txs.append(txk.reshape(EK))
        tys.append(tyk.reshape(EK))
    txa = jnp.concatenate([txs[0], txs[1], txs[2][:EK // 2]], axis=0)
    tya = jnp.concatenate([tys[0], tys[1], tys[2][:EK // 2]], axis=0)
    txb = jnp.concatenate([txs[2][EK // 2:], txs[3], txs[4]], axis=0)
    tyb = jnp.concatenate([tys[2][EK // 2:], tys[3], tys[4]], axis=0)
    pa = _sc_scatter(txa, tya, rows[:EH])
    pb = _sc_scatter(txb, tyb, rows[EH:])
    partials = jnp.concatenate([pa, pb], axis=0)
    psum = _psum(partials.reshape(2 * NW, 3 * (NP // 128), 128))
    f = psum.reshape(3, NP)
    ax = f[0, :N].reshape(N, 1)
    ay = f[1, :N].reshape(N, 1)
    ac = f[2, :N].reshape(N, 1)
    mu, logp = _final(ax, ay, ac, g, vel,
                      loc, h, Wm1, bm1.reshape(1, 128), Wm2,
                      bm2.reshape(1, 128), Wm3, bm3.reshape(1, 2),
                      log_std.reshape(1, 2))
    return (mu.reshape(100, 100, 2), logp.reshape(100, 100, 2))
